# Initial kernel scaffold; baseline (speedup 1.0000x reference)
import jax
import jax.numpy as jnp
from jax.experimental import pallas as pl


def kernel(pos, batch, W1, b1, g1, be1, W2, b2, g2, be2, W3, b3, Wc2, bc2, Wl, bl, Wm1, bm1, Wm2, bm2, Wm3, bm3):
    raise NotImplementedError("write your pallas kernel here")


# speedup vs baseline: 3.3419x; 3.3419x over previous
import functools

import jax
import jax.numpy as jnp
from jax import lax
from jax.experimental import pallas as pl
from jax.experimental.pallas import tpu as pltpu
from jax.experimental.pallas import tpu_sc as plsc

N = 8192
B = 8
K = 20
BIG = 1e30
BIG2 = 2e30


def _ext_body(x_ref, xc_ref, xr_ref, *, pad):
    x = x_ref[:]
    d2 = jnp.sum(x * x, axis=1, keepdims=True)
    one = jnp.ones_like(d2)
    cols = [x, one, d2]
    rows = [-2.0 * x, d2, one]
    if pad:
        z = jnp.zeros((x.shape[0], pad), jnp.float32)
        cols.append(z)
        rows.append(z)
    xc_ref[:] = jnp.concatenate(cols, axis=1)
    xr_ref[:] = jnp.concatenate(rows, axis=1)


def _ext(x, dp):
    d = x.shape[1]
    pad = dp - d - 2
    return pl.pallas_call(
        functools.partial(_ext_body, pad=pad),
        out_shape=(
            jax.ShapeDtypeStruct((N, dp), jnp.float32),
            jax.ShapeDtypeStruct((N, dp), jnp.float32),
        ),
    )(x)


_RKNN = 128


def _knn_body(xc_ref, xr_ref, bc_ref, br_ref, idx_ref, dist_ref):
    d = lax.dot_general(
        xc_ref[:], xr_ref[:],
        dimension_numbers=(((1,), (1,)), ((), ())),
        preferred_element_type=jnp.float32,
    )
    d = jnp.where(bc_ref[:] != br_ref[:], BIG, d)
    dist_ref[:] = d
    rowi = lax.broadcasted_iota(jnp.int32, (N, _RKNN), 0)
    picks = []
    for _ in range(K):
        d = dist_ref[:]
        m = jnp.min(d, axis=0, keepdims=True)
        cand = jnp.where(d == m, rowi, jnp.int32(2**30))
        ji = jnp.min(cand, axis=0, keepdims=True)
        picks.append(ji)
        dist_ref[:] = jnp.where(rowi == ji, BIG2, d)
    idx_ref[:] = jnp.concatenate(picks, axis=0)


def _knn(xc, xr, bc, br):
    dp = xc.shape[1]
    return pl.pallas_call(
        _knn_body,
        grid=(N // _RKNN,),
        in_specs=[
            pl.BlockSpec((N, dp), lambda i: (0, 0)),
            pl.BlockSpec((_RKNN, dp), lambda i: (i, 0)),
            pl.BlockSpec((N, 1), lambda i: (0, 0)),
            pl.BlockSpec((1, _RKNN), lambda i: (0, i)),
        ],
        out_specs=pl.BlockSpec((K, _RKNN), lambda i: (0, i)),
        out_shape=jax.ShapeDtypeStruct((K, N), jnp.int32),
        scratch_shapes=[pltpu.VMEM((N, _RKNN), jnp.float32)],
    )(xc, xr, bc, br)


_NC = 2
_NS = 16
_NW = _NC * _NS
_CH = 128


def _sc_gather(table, idx, d):
    e = idx.shape[0]
    per_w = e // _NW
    n_ch = per_w // _CH
    mesh = plsc.VectorSubcoreMesh(core_axis_name="c", subcore_axis_name="s")

    @functools.partial(
        pl.kernel,
        out_type=jax.ShapeDtypeStruct((e, d), jnp.float32),
        mesh=mesh,
        scratch_types=[
            pltpu.VMEM((_CH,), jnp.int32),
            pltpu.VMEM((_CH, d), jnp.float32),
            pltpu.SemaphoreType.DMA,
        ],
    )
    def gk(table_hbm, idx_hbm, out_hbm, idx_v, rows_v, sem):
        wid = lax.axis_index("s") * _NC + lax.axis_index("c")
        base = wid * per_w

        def body(t, carry):
            off = base + t * _CH
            pltpu.sync_copy(idx_hbm.at[pl.ds(off, _CH)], idx_v)
            pltpu.async_copy(table_hbm.at[idx_v], rows_v, sem).wait()
            pltpu.sync_copy(rows_v, out_hbm.at[pl.ds(off, _CH)])
            return carry

        lax.fori_loop(0, n_ch, body, 0)

    return gk(table, idx)


_RMLP = 512
_E1 = K * N


def _ag_body(p_ref, wd_ref, wb_ref, b1_ref, a_ref, g_ref):
    p = p_ref[:]
    a_ref[:] = jnp.dot(p, wd_ref[:], preferred_element_type=jnp.float32) + b1_ref[:]
    g_ref[:] = jnp.dot(p, wb_ref[:], preferred_element_type=jnp.float32)


def _ag(pos, wd, wb, b1):
    return pl.pallas_call(
        _ag_body,
        out_shape=(
            jax.ShapeDtypeStruct((N, 64), jnp.float32),
            jax.ShapeDtypeStruct((N, 128), jnp.float32),
        ),
    )(pos, wd, jnp.concatenate([wb, jnp.zeros((3, 64), jnp.float32)], axis=1),
      b1.reshape(1, 64))


def _h0stats_body(g_ref, a_ref, s_ref):
    @pl.when(pl.program_id(0) == 0)
    def _():
        s_ref[:] = jnp.zeros_like(s_ref)

    h0 = g_ref[:, :, 0:64] + a_ref[:][None]
    t = jnp.sum(h0, axis=0)
    u = jnp.sum(h0 * h0, axis=0)
    s_ref[0:1, :] += jnp.sum(t, axis=0, keepdims=True)
    s_ref[1:2, :] += jnp.sum(u, axis=0, keepdims=True)


def _h0stats(g3, a):
    return pl.pallas_call(
        _h0stats_body,
        grid=(N // _RMLP,),
        in_specs=[
            pl.BlockSpec((K, _RMLP, 128), lambda i: (0, i, 0)),
            pl.BlockSpec((_RMLP, 64), lambda i: (i, 0)),
        ],
        out_specs=pl.BlockSpec((8, 64), lambda i: (0, 0)),
        out_shape=jax.ShapeDtypeStruct((8, 64), jnp.float32),
    )(g3, a)


def _bn_coefs(st_ref, gamma_ref, beta_ref, count):
    mu = st_ref[0:1, :] * (1.0 / count)
    var = st_ref[1:2, :] * (1.0 / count) - mu * mu
    scale = gamma_ref[:] / jnp.sqrt(var + 1e-5)
    shift = beta_ref[:] - mu * scale
    return scale, shift


def _p2_body(g_ref, a_ref, st_ref, ga_ref, be_ref, w_ref, b_ref, h1_ref, s2_ref):
    @pl.when(pl.program_id(0) == 0)
    def _():
        s2_ref[:] = jnp.zeros_like(s2_ref)

    scale, shift = _bn_coefs(st_ref, ga_ref, be_ref, _E1)
    h0 = g_ref[:, :, 0:64] + a_ref[:][None]
    y = jnp.maximum(h0 * scale[None] + shift[None], 0.0)
    y2 = y.reshape(K * _RMLP, 64)
    h1 = jnp.dot(y2, w_ref[:], preferred_element_type=jnp.float32) + b_ref[:]
    h1_ref[:] = h1.reshape(K, _RMLP, 64)
    s2_ref[0:1, :] += jnp.sum(h1, axis=0, keepdims=True)
    s2_ref[1:2, :] += jnp.sum(h1 * h1, axis=0, keepdims=True)


def _p2(g3, a, st, gamma, beta, w2, b2):
    return pl.pallas_call(
        _p2_body,
        grid=(N // _RMLP,),
        in_specs=[
            pl.BlockSpec((K, _RMLP, 128), lambda i: (0, i, 0)),
            pl.BlockSpec((_RMLP, 64), lambda i: (i, 0)),
            pl.BlockSpec((8, 64), lambda i: (0, 0)),
            pl.BlockSpec((1, 64), lambda i: (0, 0)),
            pl.BlockSpec((1, 64), lambda i: (0, 0)),
            pl.BlockSpec((64, 64), lambda i: (0, 0)),
            pl.BlockSpec((1, 64), lambda i: (0, 0)),
        ],
        out_specs=(
            pl.BlockSpec((K, _RMLP, 64), lambda i: (0, i, 0)),
            pl.BlockSpec((8, 64), lambda i: (0, 0)),
        ),
        out_shape=(
            jax.ShapeDtypeStruct((K, N, 64), jnp.float32),
            jax.ShapeDtypeStruct((8, 64), jnp.float32),
        ),
    )(g3, a, st, gamma.reshape(1, 64), beta.reshape(1, 64), w2, b2.reshape(1, 64))


def _p3_body(h1_ref, st_ref, ga_ref, be_ref, w_ref, b_ref, x1_ref):
    scale, shift = _bn_coefs(st_ref, ga_ref, be_ref, _E1)
    y = jnp.maximum(h1_ref[:] * scale[None] + shift[None], 0.0)
    y2 = y.reshape(K * _RMLP, 64)
    h2 = jnp.dot(y2, w_ref[:], preferred_element_type=jnp.float32) + b_ref[:]
    x1_ref[:] = jnp.max(h2.reshape(K, _RMLP, 64), axis=0)


def _p3(h1, st, gamma, beta, w3, b3):
    return pl.pallas_call(
        _p3_body,
        grid=(N // _RMLP,),
        in_specs=[
            pl.BlockSpec((K, _RMLP, 64), lambda i: (0, i, 0)),
            pl.BlockSpec((8, 64), lambda i: (0, 0)),
            pl.BlockSpec((1, 64), lambda i: (0, 0)),
            pl.BlockSpec((1, 64), lambda i: (0, 0)),
            pl.BlockSpec((64, 64), lambda i: (0, 0)),
            pl.BlockSpec((1, 64), lambda i: (0, 0)),
        ],
        out_specs=pl.BlockSpec((_RMLP, 64), lambda i: (i, 0)),
        out_shape=jax.ShapeDtypeStruct((N, 64), jnp.float32),
    )(h1, st, gamma.reshape(1, 64), beta.reshape(1, 64), w3, b3.reshape(1, 64))


def _yz_body(x_ref, wz_ref, bz_ref, wy_ref, z_ref, y_ref):
    x = x_ref[:]
    z_ref[:] = jnp.dot(x, wz_ref[:], preferred_element_type=jnp.float32) + bz_ref[:]
    y_ref[:] = jnp.dot(x, wy_ref[:], preferred_element_type=jnp.float32)


def _yz(x1, wz, bz, wy):
    return pl.pallas_call(
        _yz_body,
        out_shape=(
            jax.ShapeDtypeStruct((N, 128), jnp.float32),
            jax.ShapeDtypeStruct((N, 128), jnp.float32),
        ),
    )(x1, wz, bz.reshape(1, 128), wy)


def _aggmax_body(y_ref, z_ref, x2_ref):
    x2_ref[:] = z_ref[:] + jnp.max(y_ref[:], axis=0)


def _aggmax(y3, z):
    return pl.pallas_call(
        _aggmax_body,
        grid=(N // _RMLP,),
        in_specs=[
            pl.BlockSpec((K, _RMLP, 128), lambda i: (0, i, 0)),
            pl.BlockSpec((_RMLP, 128), lambda i: (i, 0)),
        ],
        out_specs=pl.BlockSpec((_RMLP, 128), lambda i: (i, 0)),
        out_shape=jax.ShapeDtypeStruct((N, 128), jnp.float32),
    )(y3, z)


def _pool_body(x1_ref, x2_ref, bt_ref, wl_ref, bl_ref, p_ref):
    @pl.when(pl.program_id(0) == 0)
    def _():
        p_ref[:] = jnp.full_like(p_ref, -jnp.inf)

    xc = jnp.concatenate([x1_ref[:], x2_ref[:]], axis=1)
    out = jnp.dot(xc, wl_ref[:], preferred_element_type=jnp.float32) + bl_ref[:]
    bt = bt_ref[:]
    for b in range(B):
        mb = jnp.max(jnp.where(bt == b, out, -jnp.inf), axis=0, keepdims=True)
        p_ref[pl.ds(b, 1), :] = jnp.maximum(p_ref[pl.ds(b, 1), :], mb)


def _pool(x1, x2, bc, wl, bl):
    return pl.pallas_call(
        _pool_body,
        grid=(N // _RMLP,),
        in_specs=[
            pl.BlockSpec((_RMLP, 64), lambda i: (i, 0)),
            pl.BlockSpec((_RMLP, 128), lambda i: (i, 0)),
            pl.BlockSpec((_RMLP, 1), lambda i: (i, 0)),
            pl.BlockSpec((192, 1024), lambda i: (0, 0)),
            pl.BlockSpec((1, 1024), lambda i: (0, 0)),
        ],
        out_specs=pl.BlockSpec((B, 1024), lambda i: (0, 0)),
        out_shape=jax.ShapeDtypeStruct((B, 1024), jnp.float32),
    )(x1, x2, bc, wl, bl.reshape(1, 1024))


def _head_body(p_ref, w1_ref, b1_ref, w2_ref, b2_ref, w3_ref, b3_ref, o_ref):
    h = jnp.maximum(
        jnp.dot(p_ref[:], w1_ref[:], preferred_element_type=jnp.float32) + b1_ref[:], 0.0)
    h = jnp.maximum(
        jnp.dot(h, w2_ref[:], preferred_element_type=jnp.float32) + b2_ref[:], 0.0)
    h = jnp.dot(h, w3_ref[:], preferred_element_type=jnp.float32) + b3_ref[:]
    m = jnp.max(h, axis=1, keepdims=True)
    s = h - m
    o_ref[:] = s - jnp.log(jnp.sum(jnp.exp(s), axis=1, keepdims=True))


def _head(p, wm1, bm1, wm2, bm2, wm3, bm3):
    return pl.pallas_call(
        _head_body,
        out_shape=jax.ShapeDtypeStruct((B, 40), jnp.float32),
    )(p, wm1, bm1.reshape(1, 512), wm2, bm2.reshape(1, 256), wm3, bm3.reshape(1, 40))


def kernel(pos, batch, W1, b1, g1, be1, W2, b2, g2, be2, W3, b3, Wc2, bc2,
           Wl, bl, Wm1, bm1, Wm2, bm2, Wm3, bm3):
    batch = batch.astype(jnp.int32)
    bc = batch.reshape(N, 1)
    br = batch.reshape(1, N)

    xc1, xr1 = _ext(pos, 8)
    idx1 = _knn(xc1, xr1, bc, br)
    a1, g1t = _ag(pos, W1[:3] - W1[3:], W1[3:], b1)
    gat = _sc_gather(g1t, idx1.reshape(K * N), 128)
    g3 = gat.reshape(K, N, 128)
    st1 = _h0stats(g3, a1)
    h1, st2 = _p2(g3, a1, st1, g1, be1, W2, b2)
    x1 = _p3(h1, st2, g2, be2, W3, b3)

    xc2, xr2 = _ext(x1, 72)
    idx2 = _knn(xc2, xr2, bc, br)
    z, y = _yz(x1, Wc2[:64] - Wc2[64:], bc2, Wc2[64:])
    yg = _sc_gather(y, idx2.reshape(K * N), 128)
    x2 = _aggmax(yg.reshape(K, N, 128), z)

    pooled = _pool(x1, x2, bc, Wl, bl)
    return _head(pooled, Wm1, bm1, Wm2, bm2, Wm3, bm3)


# speedup vs baseline: 12.8052x; 3.8317x over previous
import functools

import jax
import jax.numpy as jnp
from jax import lax
from jax.experimental import pallas as pl
from jax.experimental.pallas import tpu as pltpu
from jax.experimental.pallas import tpu_sc as plsc

N = 8192
B = 8
K = 20
BIG = 1e30
BIG2 = 2e30


def _ext_body(x_ref, xc_ref, xr_ref, *, pad):
    x = x_ref[:]
    d2 = jnp.sum(x * x, axis=1, keepdims=True)
    one = jnp.ones_like(d2)
    cols = [x, one, d2]
    rows = [-2.0 * x, d2, one]
    if pad:
        z = jnp.zeros((x.shape[0], pad), jnp.float32)
        cols.append(z)
        rows.append(z)
    xc_ref[:] = jnp.concatenate(cols, axis=1)
    xr_ref[:] = jnp.concatenate(rows, axis=1)


def _ext(x, dp):
    d = x.shape[1]
    pad = dp - d - 2
    return pl.pallas_call(
        functools.partial(_ext_body, pad=pad),
        out_shape=(
            jax.ShapeDtypeStruct((N, dp), jnp.float32),
            jax.ShapeDtypeStruct((N, dp), jnp.float32),
        ),
    )(x)


_RKNN = 128
_CW = 512
_IBIG = 2**30


def _knn_body(c0_ref, nch_ref, xc_ref, xr_ref, bc_ref, br_ref, idx_ref,
              dist_ref):
    i = pl.program_id(0)
    c0 = c0_ref[i]
    nch = nch_ref[i]
    br = br_ref[:]
    xr = xr_ref[:]
    rowi0 = lax.broadcasted_iota(jnp.int32, (_CW, _RKNN), 0)

    def fill(t, carry):
        start = c0 + t * _CW
        d = lax.dot_general(
            xc_ref[pl.ds(start, _CW), :], xr,
            dimension_numbers=(((1,), (1,)), ((), ())),
            preferred_element_type=jnp.float32,
        )
        d = jnp.where(bc_ref[pl.ds(start, _CW), :] != br, BIG, d)
        dist_ref[pl.ds(t * _CW, _CW), :] = d
        return carry

    lax.fori_loop(0, nch, fill, 0)

    picks = []
    for _ in range(K):
        def amin(t, m):
            d = dist_ref[pl.ds(t * _CW, _CW), :]
            return jnp.minimum(m, jnp.min(d, axis=0, keepdims=True))

        m = lax.fori_loop(0, nch, amin,
                          jnp.full((1, _RKNN), jnp.inf, jnp.float32))

        def aidx(t, ji):
            d = dist_ref[pl.ds(t * _CW, _CW), :]
            cand = jnp.where(d == m, rowi0 + (c0 + t * _CW), jnp.int32(_IBIG))
            return jnp.minimum(ji, jnp.min(cand, axis=0, keepdims=True))

        ji = lax.fori_loop(0, nch, aidx, jnp.full((1, _RKNN), _IBIG,
                                                  jnp.int32))
        picks.append(ji)

        def mask(t, carry):
            d = dist_ref[pl.ds(t * _CW, _CW), :]
            sel = (rowi0 + (c0 + t * _CW)) == ji
            dist_ref[pl.ds(t * _CW, _CW), :] = jnp.where(sel, BIG2, d)
            return carry

        lax.fori_loop(0, nch, mask, 0)
    idx_ref[:] = jnp.concatenate(picks, axis=0)


def _knn(xc, xr, bc, br, c0s, nchs):
    dp = xc.shape[1]
    grid_spec = pltpu.PrefetchScalarGridSpec(
        num_scalar_prefetch=2,
        grid=(N // _RKNN,),
        in_specs=[
            pl.BlockSpec((N, dp), lambda i, *_: (0, 0)),
            pl.BlockSpec((_RKNN, dp), lambda i, *_: (i, 0)),
            pl.BlockSpec((N, 1), lambda i, *_: (0, 0)),
            pl.BlockSpec((1, _RKNN), lambda i, *_: (0, i)),
        ],
        out_specs=pl.BlockSpec((K, _RKNN), lambda i, *_: (0, i)),
        scratch_shapes=[pltpu.VMEM((N, _RKNN), jnp.float32)],
    )
    return pl.pallas_call(
        _knn_body,
        grid_spec=grid_spec,
        out_shape=jax.ShapeDtypeStruct((K, N), jnp.int32),
    )(c0s, nchs, xc, xr, bc, br)


def _seg_ranges(batch):
    ar = jnp.arange(B, dtype=batch.dtype)
    seg_start = jnp.searchsorted(batch, ar).astype(jnp.int32)
    seg_end = jnp.searchsorted(batch, ar, side="right").astype(jnp.int32)
    sizes = seg_end - seg_start
    degenerate = jnp.any((sizes > 0) & (sizes < K))
    b2 = batch.reshape(N // _RKNN, _RKNN)
    first_b = b2[:, 0]
    last_b = b2[:, -1]
    c0 = (seg_start[first_b] // _CW) * _CW
    nch = (seg_end[last_b] - c0 + _CW - 1) // _CW
    c0 = jnp.where(degenerate, 0, c0).astype(jnp.int32)
    nch = jnp.where(degenerate, N // _CW, nch).astype(jnp.int32)
    return c0, nch


_NC = 2
_NS = 16
_NW = _NC * _NS
_CH = 128


def _sc_gather(table, idx, d):
    e = idx.shape[0]
    per_w = e // _NW
    n_ch = per_w // _CH
    mesh = plsc.VectorSubcoreMesh(core_axis_name="c", subcore_axis_name="s")

    @functools.partial(
        pl.kernel,
        out_type=jax.ShapeDtypeStruct((e, d), jnp.float32),
        mesh=mesh,
        scratch_types=[
            pltpu.VMEM((_CH,), jnp.int32),
            pltpu.VMEM((_CH, d), jnp.float32),
            pltpu.SemaphoreType.DMA,
        ],
    )
    def gk(table_hbm, idx_hbm, out_hbm, idx_v, rows_v, sem):
        wid = lax.axis_index("s") * _NC + lax.axis_index("c")
        base = wid * per_w

        def body(t, carry):
            off = base + t * _CH
            pltpu.sync_copy(idx_hbm.at[pl.ds(off, _CH)], idx_v)
            pltpu.async_copy(table_hbm.at[idx_v], rows_v, sem).wait()
            pltpu.sync_copy(rows_v, out_hbm.at[pl.ds(off, _CH)])
            return carry

        lax.fori_loop(0, n_ch, body, 0)

    return gk(table, idx)


_RMLP = 512
_E1 = K * N


def _ag_body(p_ref, wd_ref, wb_ref, b1_ref, a_ref, g_ref):
    p = p_ref[:]
    a_ref[:] = jnp.dot(p, wd_ref[:], preferred_element_type=jnp.float32) + b1_ref[:]
    g_ref[:] = jnp.dot(p, wb_ref[:], preferred_element_type=jnp.float32)


def _ag(pos, wd, wb, b1):
    return pl.pallas_call(
        _ag_body,
        out_shape=(
            jax.ShapeDtypeStruct((N, 64), jnp.float32),
            jax.ShapeDtypeStruct((N, 128), jnp.float32),
        ),
    )(pos, wd, jnp.concatenate([wb, jnp.zeros((3, 64), jnp.float32)], axis=1),
      b1.reshape(1, 64))


def _h0stats_body(g_ref, a_ref, s_ref):
    @pl.when(pl.program_id(0) == 0)
    def _():
        s_ref[:] = jnp.zeros_like(s_ref)

    h0 = g_ref[:, :, 0:64] + a_ref[:][None]
    t = jnp.sum(h0, axis=0)
    u = jnp.sum(h0 * h0, axis=0)
    s_ref[0:1, :] += jnp.sum(t, axis=0, keepdims=True)
    s_ref[1:2, :] += jnp.sum(u, axis=0, keepdims=True)


def _h0stats(g3, a):
    return pl.pallas_call(
        _h0stats_body,
        grid=(N // _RMLP,),
        in_specs=[
            pl.BlockSpec((K, _RMLP, 128), lambda i: (0, i, 0)),
            pl.BlockSpec((_RMLP, 64), lambda i: (i, 0)),
        ],
        out_specs=pl.BlockSpec((8, 64), lambda i: (0, 0)),
        out_shape=jax.ShapeDtypeStruct((8, 64), jnp.float32),
    )(g3, a)


def _bn_coefs(st_ref, gamma_ref, beta_ref, count):
    mu = st_ref[0:1, :] * (1.0 / count)
    var = st_ref[1:2, :] * (1.0 / count) - mu * mu
    scale = gamma_ref[:] / jnp.sqrt(var + 1e-5)
    shift = beta_ref[:] - mu * scale
    return scale, shift


def _p2_body(g_ref, a_ref, st_ref, ga_ref, be_ref, w_ref, b_ref, h1_ref, s2_ref):
    @pl.when(pl.program_id(0) == 0)
    def _():
        s2_ref[:] = jnp.zeros_like(s2_ref)

    scale, shift = _bn_coefs(st_ref, ga_ref, be_ref, _E1)
    h0 = g_ref[:, :, 0:64] + a_ref[:][None]
    y = jnp.maximum(h0 * scale[None] + shift[None], 0.0)
    y2 = y.reshape(K * _RMLP, 64)
    h1 = jnp.dot(y2, w_ref[:], preferred_element_type=jnp.float32) + b_ref[:]
    h1_ref[:] = h1.reshape(K, _RMLP, 64)
    s2_ref[0:1, :] += jnp.sum(h1, axis=0, keepdims=True)
    s2_ref[1:2, :] += jnp.sum(h1 * h1, axis=0, keepdims=True)


def _p2(g3, a, st, gamma, beta, w2, b2):
    return pl.pallas_call(
        _p2_body,
        grid=(N // _RMLP,),
        in_specs=[
            pl.BlockSpec((K, _RMLP, 128), lambda i: (0, i, 0)),
            pl.BlockSpec((_RMLP, 64), lambda i: (i, 0)),
            pl.BlockSpec((8, 64), lambda i: (0, 0)),
            pl.BlockSpec((1, 64), lambda i: (0, 0)),
            pl.BlockSpec((1, 64), lambda i: (0, 0)),
            pl.BlockSpec((64, 64), lambda i: (0, 0)),
            pl.BlockSpec((1, 64), lambda i: (0, 0)),
        ],
        out_specs=(
            pl.BlockSpec((K, _RMLP, 64), lambda i: (0, i, 0)),
            pl.BlockSpec((8, 64), lambda i: (0, 0)),
        ),
        out_shape=(
            jax.ShapeDtypeStruct((K, N, 64), jnp.float32),
            jax.ShapeDtypeStruct((8, 64), jnp.float32),
        ),
    )(g3, a, st, gamma.reshape(1, 64), beta.reshape(1, 64), w2, b2.reshape(1, 64))


def _p3_body(h1_ref, st_ref, ga_ref, be_ref, w_ref, b_ref, x1_ref):
    scale, shift = _bn_coefs(st_ref, ga_ref, be_ref, _E1)
    y = jnp.maximum(h1_ref[:] * scale[None] + shift[None], 0.0)
    y2 = y.reshape(K * _RMLP, 64)
    h2 = jnp.dot(y2, w_ref[:], preferred_element_type=jnp.float32) + b_ref[:]
    x1_ref[:] = jnp.max(h2.reshape(K, _RMLP, 64), axis=0)


def _p3(h1, st, gamma, beta, w3, b3):
    return pl.pallas_call(
        _p3_body,
        grid=(N // _RMLP,),
        in_specs=[
            pl.BlockSpec((K, _RMLP, 64), lambda i: (0, i, 0)),
            pl.BlockSpec((8, 64), lambda i: (0, 0)),
            pl.BlockSpec((1, 64), lambda i: (0, 0)),
            pl.BlockSpec((1, 64), lambda i: (0, 0)),
            pl.BlockSpec((64, 64), lambda i: (0, 0)),
            pl.BlockSpec((1, 64), lambda i: (0, 0)),
        ],
        out_specs=pl.BlockSpec((_RMLP, 64), lambda i: (i, 0)),
        out_shape=jax.ShapeDtypeStruct((N, 64), jnp.float32),
    )(h1, st, gamma.reshape(1, 64), beta.reshape(1, 64), w3, b3.reshape(1, 64))


def _yz_body(x_ref, wz_ref, bz_ref, wy_ref, z_ref, y_ref):
    x = x_ref[:]
    z_ref[:] = jnp.dot(x, wz_ref[:], preferred_element_type=jnp.float32) + bz_ref[:]
    y_ref[:] = jnp.dot(x, wy_ref[:], preferred_element_type=jnp.float32)


def _yz(x1, wz, bz, wy):
    return pl.pallas_call(
        _yz_body,
        out_shape=(
            jax.ShapeDtypeStruct((N, 128), jnp.float32),
            jax.ShapeDtypeStruct((N, 128), jnp.float32),
        ),
    )(x1, wz, bz.reshape(1, 128), wy)


def _aggmax_body(y_ref, z_ref, x2_ref):
    x2_ref[:] = z_ref[:] + jnp.max(y_ref[:], axis=0)


def _aggmax(y3, z):
    return pl.pallas_call(
        _aggmax_body,
        grid=(N // _RMLP,),
        in_specs=[
            pl.BlockSpec((K, _RMLP, 128), lambda i: (0, i, 0)),
            pl.BlockSpec((_RMLP, 128), lambda i: (i, 0)),
        ],
        out_specs=pl.BlockSpec((_RMLP, 128), lambda i: (i, 0)),
        out_shape=jax.ShapeDtypeStruct((N, 128), jnp.float32),
    )(y3, z)


def _pool_body(x1_ref, x2_ref, bt_ref, wl_ref, bl_ref, p_ref):
    @pl.when(pl.program_id(0) == 0)
    def _():
        p_ref[:] = jnp.full_like(p_ref, -jnp.inf)

    xc = jnp.concatenate([x1_ref[:], x2_ref[:]], axis=1)
    out = jnp.dot(xc, wl_ref[:], preferred_element_type=jnp.float32) + bl_ref[:]
    bt = bt_ref[:]
    for b in range(B):
        mb = jnp.max(jnp.where(bt == b, out, -jnp.inf), axis=0, keepdims=True)
        p_ref[pl.ds(b, 1), :] = jnp.maximum(p_ref[pl.ds(b, 1), :], mb)


def _pool(x1, x2, bc, wl, bl):
    return pl.pallas_call(
        _pool_body,
        grid=(N // _RMLP,),
        in_specs=[
            pl.BlockSpec((_RMLP, 64), lambda i: (i, 0)),
            pl.BlockSpec((_RMLP, 128), lambda i: (i, 0)),
            pl.BlockSpec((_RMLP, 1), lambda i: (i, 0)),
            pl.BlockSpec((192, 1024), lambda i: (0, 0)),
            pl.BlockSpec((1, 1024), lambda i: (0, 0)),
        ],
        out_specs=pl.BlockSpec((B, 1024), lambda i: (0, 0)),
        out_shape=jax.ShapeDtypeStruct((B, 1024), jnp.float32),
    )(x1, x2, bc, wl, bl.reshape(1, 1024))


def _head_body(p_ref, w1_ref, b1_ref, w2_ref, b2_ref, w3_ref, b3_ref, o_ref):
    h = jnp.maximum(
        jnp.dot(p_ref[:], w1_ref[:], preferred_element_type=jnp.float32) + b1_ref[:], 0.0)
    h = jnp.maximum(
        jnp.dot(h, w2_ref[:], preferred_element_type=jnp.float32) + b2_ref[:], 0.0)
    h = jnp.dot(h, w3_ref[:], preferred_element_type=jnp.float32) + b3_ref[:]
    m = jnp.max(h, axis=1, keepdims=True)
    s = h - m
    o_ref[:] = s - jnp.log(jnp.sum(jnp.exp(s), axis=1, keepdims=True))


def _head(p, wm1, bm1, wm2, bm2, wm3, bm3):
    return pl.pallas_call(
        _head_body,
        out_shape=jax.ShapeDtypeStruct((B, 40), jnp.float32),
    )(p, wm1, bm1.reshape(1, 512), wm2, bm2.reshape(1, 256), wm3, bm3.reshape(1, 40))


def kernel(pos, batch, W1, b1, g1, be1, W2, b2, g2, be2, W3, b3, Wc2, bc2,
           Wl, bl, Wm1, bm1, Wm2, bm2, Wm3, bm3):
    batch = batch.astype(jnp.int32)
    bc = batch.reshape(N, 1)
    br = batch.reshape(1, N)
    c0s, nchs = _seg_ranges(batch)

    xc1, xr1 = _ext(pos, 8)
    idx1 = _knn(xc1, xr1, bc, br, c0s, nchs)
    a1, g1t = _ag(pos, W1[:3] - W1[3:], W1[3:], b1)
    gat = _sc_gather(g1t, idx1.reshape(K * N), 128)
    g3 = gat.reshape(K, N, 128)
    st1 = _h0stats(g3, a1)
    h1, st2 = _p2(g3, a1, st1, g1, be1, W2, b2)
    x1 = _p3(h1, st2, g2, be2, W3, b3)

    xc2, xr2 = _ext(x1, 72)
    idx2 = _knn(xc2, xr2, bc, br, c0s, nchs)
    z, y = _yz(x1, Wc2[:64] - Wc2[64:], bc2, Wc2[64:])
    yg = _sc_gather(y, idx2.reshape(K * N), 128)
    x2 = _aggmax(yg.reshape(K, N, 128), z)

    pooled = _pool(x1, x2, bc, Wl, bl)
    return _head(pooled, Wm1, bm1, Wm2, bm2, Wm3, bm3)


# speedup vs baseline: 14.5808x; 1.1387x over previous
import functools

import jax
import jax.numpy as jnp
from jax import lax
from jax.experimental import pallas as pl
from jax.experimental.pallas import tpu as pltpu
from jax.experimental.pallas import tpu_sc as plsc

N = 8192
B = 8
K = 20
BIG = 1e30
BIG2 = 2e30


def _ext_body(x_ref, xc_ref, xr_ref, *, pad):
    x = x_ref[:]
    d2 = jnp.sum(x * x, axis=1, keepdims=True)
    one = jnp.ones_like(d2)
    cols = [x, one, d2]
    rows = [-2.0 * x, d2, one]
    if pad:
        z = jnp.zeros((x.shape[0], pad), jnp.float32)
        cols.append(z)
        rows.append(z)
    xc_ref[:] = jnp.concatenate(cols, axis=1)
    xr_ref[:] = jnp.concatenate(rows, axis=1)


def _ext(x, dp):
    d = x.shape[1]
    pad = dp - d - 2
    return pl.pallas_call(
        functools.partial(_ext_body, pad=pad),
        out_shape=(
            jax.ShapeDtypeStruct((N, dp), jnp.float32),
            jax.ShapeDtypeStruct((N, dp), jnp.float32),
        ),
    )(x)


_RKNN = 128
_CW = 512
_IBIG = 2**30


def _knn_body(c0_ref, nch_ref, xc_ref, xr_ref, bc_ref, br_ref, idx_ref,
              dist_ref):
    i = pl.program_id(0)
    c0 = c0_ref[i]
    nch = nch_ref[i]
    br = br_ref[:]
    xr = xr_ref[:]
    rowi0 = lax.broadcasted_iota(jnp.int32, (_CW, _RKNN), 0)

    def fill(t, carry):
        start = c0 + t * _CW
        d = lax.dot_general(
            xc_ref[pl.ds(start, _CW), :], xr,
            dimension_numbers=(((1,), (1,)), ((), ())),
            preferred_element_type=jnp.float32,
        )
        d = jnp.where(bc_ref[pl.ds(start, _CW), :] != br, BIG, d)
        dist_ref[pl.ds(t * _CW, _CW), :] = d
        return carry

    lax.fori_loop(0, nch, fill, 0)

    def amin(t, m):
        d = dist_ref[pl.ds(t * _CW, _CW), :]
        return jnp.minimum(m, jnp.min(d, axis=0, keepdims=True))

    m = lax.fori_loop(0, nch, amin,
                      jnp.full((1, _RKNN), jnp.inf, jnp.float32))
    picks = []
    for step in range(K):
        def aidx(t, ji):
            d = dist_ref[pl.ds(t * _CW, _CW), :]
            cand = jnp.where(d == m, rowi0 + (c0 + t * _CW), jnp.int32(_IBIG))
            return jnp.minimum(ji, jnp.min(cand, axis=0, keepdims=True))

        ji = lax.fori_loop(0, nch, aidx, jnp.full((1, _RKNN), _IBIG,
                                                  jnp.int32))
        picks.append(ji)

        if step < K - 1:
            def fuse(t, m2):
                d = dist_ref[pl.ds(t * _CW, _CW), :]
                d = jnp.where((rowi0 + (c0 + t * _CW)) == ji, BIG2, d)
                dist_ref[pl.ds(t * _CW, _CW), :] = d
                return jnp.minimum(m2, jnp.min(d, axis=0, keepdims=True))

            m = lax.fori_loop(0, nch, fuse,
                              jnp.full((1, _RKNN), jnp.inf, jnp.float32))
    idx_ref[:] = jnp.concatenate(picks, axis=0)


def _knn(xc, xr, bc, br, c0s, nchs):
    dp = xc.shape[1]
    grid_spec = pltpu.PrefetchScalarGridSpec(
        num_scalar_prefetch=2,
        grid=(N // _RKNN,),
        in_specs=[
            pl.BlockSpec((N, dp), lambda i, *_: (0, 0)),
            pl.BlockSpec((_RKNN, dp), lambda i, *_: (i, 0)),
            pl.BlockSpec((N, 1), lambda i, *_: (0, 0)),
            pl.BlockSpec((1, _RKNN), lambda i, *_: (0, i)),
        ],
        out_specs=pl.BlockSpec((K, _RKNN), lambda i, *_: (0, i)),
        scratch_shapes=[pltpu.VMEM((N, _RKNN), jnp.float32)],
    )
    return pl.pallas_call(
        _knn_body,
        grid_spec=grid_spec,
        out_shape=jax.ShapeDtypeStruct((K, N), jnp.int32),
    )(c0s, nchs, xc, xr, bc, br)


def _seg_ranges(batch):
    ar = jnp.arange(B, dtype=batch.dtype)
    seg_start = jnp.searchsorted(batch, ar).astype(jnp.int32)
    seg_end = jnp.searchsorted(batch, ar, side="right").astype(jnp.int32)
    sizes = seg_end - seg_start
    degenerate = jnp.any((sizes > 0) & (sizes < K))
    b2 = batch.reshape(N // _RKNN, _RKNN)
    first_b = b2[:, 0]
    last_b = b2[:, -1]
    c0 = (seg_start[first_b] // _CW) * _CW
    nch = (seg_end[last_b] - c0 + _CW - 1) // _CW
    c0 = jnp.where(degenerate, 0, c0).astype(jnp.int32)
    nch = jnp.where(degenerate, N // _CW, nch).astype(jnp.int32)
    return c0, nch


_NC = 2
_NS = 16
_NW = _NC * _NS
_CH = 256


def _sc_gather(table, idx, d):
    e = idx.shape[0]
    per_w = e // _NW
    n_ch = per_w // _CH
    mesh = plsc.VectorSubcoreMesh(core_axis_name="c", subcore_axis_name="s")

    @functools.partial(
        pl.kernel,
        out_type=jax.ShapeDtypeStruct((e, d), jnp.float32),
        mesh=mesh,
        scratch_types=[
            pltpu.VMEM((_CH,), jnp.int32),
            pltpu.VMEM((_CH,), jnp.int32),
            pltpu.VMEM((_CH, d), jnp.float32),
            pltpu.VMEM((_CH, d), jnp.float32),
            pltpu.SemaphoreType.DMA,
            pltpu.SemaphoreType.DMA,
        ],
    )
    def gk(table_hbm, idx_hbm, out_hbm, idx0, idx1, rows0, rows1, gsem, osem):
        wid = lax.axis_index("s") * _NC + lax.axis_index("c")
        base = wid * per_w

        def body(t2, carry):
            for b, (iv, rv) in enumerate(((idx0, rows0), (idx1, rows1))):
                off = base + (t2 * 2 + b) * _CH

                @pl.when(t2 >= 1)
                def _(rv=rv, off=off):
                    pltpu.make_async_copy(
                        rv, out_hbm.at[pl.ds(off, _CH)], osem).wait()

                pltpu.sync_copy(idx_hbm.at[pl.ds(off, _CH)], iv)
                pltpu.async_copy(table_hbm.at[iv], rv, gsem).wait()
                pltpu.async_copy(rv, out_hbm.at[pl.ds(off, _CH)], osem)
            return carry

        lax.fori_loop(0, n_ch // 2, body, 0)
        pltpu.make_async_copy(rows0, out_hbm.at[pl.ds(base, _CH)],
                              osem).wait()
        pltpu.make_async_copy(rows1, out_hbm.at[pl.ds(base, _CH)],
                              osem).wait()

    return gk(table, idx)


_RMLP = 512
_E1 = K * N


def _ag_body(p_ref, wd_ref, wb_ref, b1_ref, a_ref, g_ref):
    p = p_ref[:]
    a_ref[:] = jnp.dot(p, wd_ref[:], preferred_element_type=jnp.float32) + b1_ref[:]
    g_ref[:] = jnp.dot(p, wb_ref[:], preferred_element_type=jnp.float32)


def _ag(pos, wd, wb, b1):
    return pl.pallas_call(
        _ag_body,
        out_shape=(
            jax.ShapeDtypeStruct((N, 64), jnp.float32),
            jax.ShapeDtypeStruct((N, 128), jnp.float32),
        ),
    )(pos, wd, jnp.concatenate([wb, jnp.zeros((3, 64), jnp.float32)], axis=1),
      b1.reshape(1, 64))


def _h0stats_body(g_ref, a_ref, s_ref):
    @pl.when(pl.program_id(0) == 0)
    def _():
        s_ref[:] = jnp.zeros_like(s_ref)

    h0 = g_ref[:, :, 0:64] + a_ref[:][None]
    t = jnp.sum(h0, axis=0)
    u = jnp.sum(h0 * h0, axis=0)
    s_ref[0:1, :] += jnp.sum(t, axis=0, keepdims=True)
    s_ref[1:2, :] += jnp.sum(u, axis=0, keepdims=True)


def _h0stats(g3, a):
    return pl.pallas_call(
        _h0stats_body,
        grid=(N // _RMLP,),
        in_specs=[
            pl.BlockSpec((K, _RMLP, 128), lambda i: (0, i, 0)),
            pl.BlockSpec((_RMLP, 64), lambda i: (i, 0)),
        ],
        out_specs=pl.BlockSpec((8, 64), lambda i: (0, 0)),
        out_shape=jax.ShapeDtypeStruct((8, 64), jnp.float32),
    )(g3, a)


def _bn_coefs(st_ref, gamma_ref, beta_ref, count):
    mu = st_ref[0:1, :] * (1.0 / count)
    var = st_ref[1:2, :] * (1.0 / count) - mu * mu
    scale = gamma_ref[:] / jnp.sqrt(var + 1e-5)
    shift = beta_ref[:] - mu * scale
    return scale, shift


def _p2_body(g_ref, a_ref, st_ref, ga_ref, be_ref, w_ref, b_ref, h1_ref, s2_ref):
    @pl.when(pl.program_id(0) == 0)
    def _():
        s2_ref[:] = jnp.zeros_like(s2_ref)

    scale, shift = _bn_coefs(st_ref, ga_ref, be_ref, _E1)
    h0 = g_ref[:, :, 0:64] + a_ref[:][None]
    y = jnp.maximum(h0 * scale[None] + shift[None], 0.0)
    y2 = y.reshape(K * _RMLP, 64)
    h1 = jnp.dot(y2, w_ref[:], preferred_element_type=jnp.float32) + b_ref[:]
    h1_ref[:] = h1.reshape(K, _RMLP, 64)
    s2_ref[0:1, :] += jnp.sum(h1, axis=0, keepdims=True)
    s2_ref[1:2, :] += jnp.sum(h1 * h1, axis=0, keepdims=True)


def _p2(g3, a, st, gamma, beta, w2, b2):
    return pl.pallas_call(
        _p2_body,
        grid=(N // _RMLP,),
        in_specs=[
            pl.BlockSpec((K, _RMLP, 128), lambda i: (0, i, 0)),
            pl.BlockSpec((_RMLP, 64), lambda i: (i, 0)),
            pl.BlockSpec((8, 64), lambda i: (0, 0)),
            pl.BlockSpec((1, 64), lambda i: (0, 0)),
            pl.BlockSpec((1, 64), lambda i: (0, 0)),
            pl.BlockSpec((64, 64), lambda i: (0, 0)),
            pl.BlockSpec((1, 64), lambda i: (0, 0)),
        ],
        out_specs=(
            pl.BlockSpec((K, _RMLP, 64), lambda i: (0, i, 0)),
            pl.BlockSpec((8, 64), lambda i: (0, 0)),
        ),
        out_shape=(
            jax.ShapeDtypeStruct((K, N, 64), jnp.float32),
            jax.ShapeDtypeStruct((8, 64), jnp.float32),
        ),
    )(g3, a, st, gamma.reshape(1, 64), beta.reshape(1, 64), w2, b2.reshape(1, 64))


def _p3_body(h1_ref, st_ref, ga_ref, be_ref, w_ref, b_ref, x1_ref):
    scale, shift = _bn_coefs(st_ref, ga_ref, be_ref, _E1)
    y = jnp.maximum(h1_ref[:] * scale[None] + shift[None], 0.0)
    y2 = y.reshape(K * _RMLP, 64)
    h2 = jnp.dot(y2, w_ref[:], preferred_element_type=jnp.float32) + b_ref[:]
    x1_ref[:] = jnp.max(h2.reshape(K, _RMLP, 64), axis=0)


def _p3(h1, st, gamma, beta, w3, b3):
    return pl.pallas_call(
        _p3_body,
        grid=(N // _RMLP,),
        in_specs=[
            pl.BlockSpec((K, _RMLP, 64), lambda i: (0, i, 0)),
            pl.BlockSpec((8, 64), lambda i: (0, 0)),
            pl.BlockSpec((1, 64), lambda i: (0, 0)),
            pl.BlockSpec((1, 64), lambda i: (0, 0)),
            pl.BlockSpec((64, 64), lambda i: (0, 0)),
            pl.BlockSpec((1, 64), lambda i: (0, 0)),
        ],
        out_specs=pl.BlockSpec((_RMLP, 64), lambda i: (i, 0)),
        out_shape=jax.ShapeDtypeStruct((N, 64), jnp.float32),
    )(h1, st, gamma.reshape(1, 64), beta.reshape(1, 64), w3, b3.reshape(1, 64))


def _yz_body(x_ref, wz_ref, bz_ref, wy_ref, z_ref, y_ref):
    x = x_ref[:]
    z_ref[:] = jnp.dot(x, wz_ref[:], preferred_element_type=jnp.float32) + bz_ref[:]
    y_ref[:] = jnp.dot(x, wy_ref[:], preferred_element_type=jnp.float32)


def _yz(x1, wz, bz, wy):
    return pl.pallas_call(
        _yz_body,
        out_shape=(
            jax.ShapeDtypeStruct((N, 128), jnp.float32),
            jax.ShapeDtypeStruct((N, 128), jnp.float32),
        ),
    )(x1, wz, bz.reshape(1, 128), wy)


def _aggmax_body(y_ref, z_ref, x2_ref):
    x2_ref[:] = z_ref[:] + jnp.max(y_ref[:], axis=0)


def _aggmax(y3, z):
    return pl.pallas_call(
        _aggmax_body,
        grid=(N // _RMLP,),
        in_specs=[
            pl.BlockSpec((K, _RMLP, 128), lambda i: (0, i, 0)),
            pl.BlockSpec((_RMLP, 128), lambda i: (i, 0)),
        ],
        out_specs=pl.BlockSpec((_RMLP, 128), lambda i: (i, 0)),
        out_shape=jax.ShapeDtypeStruct((N, 128), jnp.float32),
    )(y3, z)


def _pool_body(x1_ref, x2_ref, bt_ref, wl_ref, bl_ref, p_ref):
    @pl.when(pl.program_id(0) == 0)
    def _():
        p_ref[:] = jnp.full_like(p_ref, -jnp.inf)

    xc = jnp.concatenate([x1_ref[:], x2_ref[:]], axis=1)
    out = jnp.dot(xc, wl_ref[:], preferred_element_type=jnp.float32) + bl_ref[:]
    bt = bt_ref[:]
    for b in range(B):
        mb = jnp.max(jnp.where(bt == b, out, -jnp.inf), axis=0, keepdims=True)
        p_ref[pl.ds(b, 1), :] = jnp.maximum(p_ref[pl.ds(b, 1), :], mb)


def _pool(x1, x2, bc, wl, bl):
    return pl.pallas_call(
        _pool_body,
        grid=(N // _RMLP,),
        in_specs=[
            pl.BlockSpec((_RMLP, 64), lambda i: (i, 0)),
            pl.BlockSpec((_RMLP, 128), lambda i: (i, 0)),
            pl.BlockSpec((_RMLP, 1), lambda i: (i, 0)),
            pl.BlockSpec((192, 1024), lambda i: (0, 0)),
            pl.BlockSpec((1, 1024), lambda i: (0, 0)),
        ],
        out_specs=pl.BlockSpec((B, 1024), lambda i: (0, 0)),
        out_shape=jax.ShapeDtypeStruct((B, 1024), jnp.float32),
    )(x1, x2, bc, wl, bl.reshape(1, 1024))


def _head_body(p_ref, w1_ref, b1_ref, w2_ref, b2_ref, w3_ref, b3_ref, o_ref):
    h = jnp.maximum(
        jnp.dot(p_ref[:], w1_ref[:], preferred_element_type=jnp.float32) + b1_ref[:], 0.0)
    h = jnp.maximum(
        jnp.dot(h, w2_ref[:], preferred_element_type=jnp.float32) + b2_ref[:], 0.0)
    h = jnp.dot(h, w3_ref[:], preferred_element_type=jnp.float32) + b3_ref[:]
    m = jnp.max(h, axis=1, keepdims=True)
    s = h - m
    o_ref[:] = s - jnp.log(jnp.sum(jnp.exp(s), axis=1, keepdims=True))


def _head(p, wm1, bm1, wm2, bm2, wm3, bm3):
    return pl.pallas_call(
        _head_body,
        out_shape=jax.ShapeDtypeStruct((B, 40), jnp.float32),
    )(p, wm1, bm1.reshape(1, 512), wm2, bm2.reshape(1, 256), wm3, bm3.reshape(1, 40))


def kernel(pos, batch, W1, b1, g1, be1, W2, b2, g2, be2, W3, b3, Wc2, bc2,
           Wl, bl, Wm1, bm1, Wm2, bm2, Wm3, bm3):
    batch = batch.astype(jnp.int32)
    bc = batch.reshape(N, 1)
    br = batch.reshape(1, N)
    c0s, nchs = _seg_ranges(batch)

    xc1, xr1 = _ext(pos, 8)
    idx1 = _knn(xc1, xr1, bc, br, c0s, nchs)
    a1, g1t = _ag(pos, W1[:3] - W1[3:], W1[3:], b1)
    gat = _sc_gather(g1t, idx1.reshape(K * N), 128)
    g3 = gat.reshape(K, N, 128)
    st1 = _h0stats(g3, a1)
    h1, st2 = _p2(g3, a1, st1, g1, be1, W2, b2)
    x1 = _p3(h1, st2, g2, be2, W3, b3)

    xc2, xr2 = _ext(x1, 72)
    idx2 = _knn(xc2, xr2, bc, br, c0s, nchs)
    z, y = _yz(x1, Wc2[:64] - Wc2[64:], bc2, Wc2[64:])
    yg = _sc_gather(y, idx2.reshape(K * N), 128)
    x2 = _aggmax(yg.reshape(K, N, 128), z)

    pooled = _pool(x1, x2, bc, Wl, bl)
    return _head(pooled, Wm1, bm1, Wm2, bm2, Wm3, bm3)


# speedup vs baseline: 18.7395x; 1.2852x over previous
import functools

import jax
import jax.numpy as jnp
from jax import lax
from jax.experimental import pallas as pl
from jax.experimental.pallas import tpu as pltpu
from jax.experimental.pallas import tpu_sc as plsc

N = 8192
B = 8
K = 20
BIG = 1e30
BIG2 = 2e30


def _ext_body(x_ref, xc_ref, xr_ref, *, pad):
    x = x_ref[:]
    d2 = jnp.sum(x * x, axis=1, keepdims=True)
    one = jnp.ones_like(d2)
    cols = [x, one, d2]
    rows = [-2.0 * x, d2, one]
    if pad:
        z = jnp.zeros((x.shape[0], pad), jnp.float32)
        cols.append(z)
        rows.append(z)
    xc_ref[:] = jnp.concatenate(cols, axis=1)
    xr_ref[:] = jnp.concatenate(rows, axis=1)


def _ext(x, dp):
    d = x.shape[1]
    pad = dp - d - 2
    return pl.pallas_call(
        functools.partial(_ext_body, pad=pad),
        out_shape=(
            jax.ShapeDtypeStruct((N, dp), jnp.float32),
            jax.ShapeDtypeStruct((N, dp), jnp.float32),
        ),
    )(x)


_RKNN = 256
_CW = 512
_IBIG = 2**30


def _knn_body(c0_ref, nch_ref, xc_ref, xr_ref, bc_ref, br_ref, idx_ref,
              dist_ref):
    i = pl.program_id(0)
    c0 = c0_ref[i]
    nch = nch_ref[i]
    br = br_ref[:]
    xr = xr_ref[:]
    rowi0 = lax.broadcasted_iota(jnp.int32, (_CW, _RKNN), 0)
    minf = jnp.full((1, _RKNN), jnp.inf, jnp.float32)
    ibig = jnp.full((1, _RKNN), _IBIG, jnp.int32)

    def _scan_chunk(d, off, m, ji):
        lm = jnp.min(d, axis=0, keepdims=True)
        la = jnp.min(jnp.where(d == lm, rowi0, jnp.int32(_IBIG)),
                     axis=0, keepdims=True) + off
        better = lm < m
        return jnp.minimum(m, lm), jnp.where(better, la, ji)

    def fill(t, mj):
        m, ji = mj
        start = c0 + t * _CW
        d = lax.dot_general(
            xc_ref[pl.ds(start, _CW), :], xr,
            dimension_numbers=(((1,), (1,)), ((), ())),
            preferred_element_type=jnp.float32,
        )
        d = jnp.where(bc_ref[pl.ds(start, _CW), :] != br, BIG, d)
        dist_ref[pl.ds(t * _CW, _CW), :] = d
        return _scan_chunk(d, start, m, ji)

    m, ji = lax.fori_loop(0, nch, fill, (minf, ibig))

    picks = [ji]
    for _ in range(1, K):
        prev = ji

        def ext(t, mj, prev=prev):
            m2, ji2 = mj
            off = c0 + t * _CW
            d = dist_ref[pl.ds(t * _CW, _CW), :]
            d = jnp.where(rowi0 == (prev - off), BIG2, d)
            dist_ref[pl.ds(t * _CW, _CW), :] = d
            return _scan_chunk(d, off, m2, ji2)

        m, ji = lax.fori_loop(0, nch, ext, (minf, ibig))
        picks.append(ji)
    idx_ref[:] = jnp.concatenate(picks, axis=0)


def _knn(xc, xr, bc, br, c0s, nchs):
    dp = xc.shape[1]
    grid_spec = pltpu.PrefetchScalarGridSpec(
        num_scalar_prefetch=2,
        grid=(N // _RKNN,),
        in_specs=[
            pl.BlockSpec((N, dp), lambda i, *_: (0, 0)),
            pl.BlockSpec((_RKNN, dp), lambda i, *_: (i, 0)),
            pl.BlockSpec((N, 1), lambda i, *_: (0, 0)),
            pl.BlockSpec((1, _RKNN), lambda i, *_: (0, i)),
        ],
        out_specs=pl.BlockSpec((K, _RKNN), lambda i, *_: (0, i)),
        scratch_shapes=[pltpu.VMEM((N, _RKNN), jnp.float32)],
    )
    return pl.pallas_call(
        _knn_body,
        grid_spec=grid_spec,
        out_shape=jax.ShapeDtypeStruct((K, N), jnp.int32),
    )(c0s, nchs, xc, xr, bc, br)


def _seg_ranges(batch):
    ar = jnp.arange(B, dtype=batch.dtype)
    seg_start = jnp.searchsorted(batch, ar).astype(jnp.int32)
    seg_end = jnp.searchsorted(batch, ar, side="right").astype(jnp.int32)
    sizes = seg_end - seg_start
    degenerate = jnp.any((sizes > 0) & (sizes < K))
    b2 = batch.reshape(N // _RKNN, _RKNN)
    first_b = b2[:, 0]
    last_b = b2[:, -1]
    c0 = (seg_start[first_b] // _CW) * _CW
    nch = (seg_end[last_b] - c0 + _CW - 1) // _CW
    c0 = jnp.where(degenerate, 0, c0).astype(jnp.int32)
    nch = jnp.where(degenerate, N // _CW, nch).astype(jnp.int32)
    return c0, nch


_NC = 2
_NS = 16
_NW = _NC * _NS
_CH = 256


def _sc_gather(table, idx, d):
    e = idx.shape[0]
    per_w = e // _NW
    n_ch = per_w // _CH
    mesh = plsc.VectorSubcoreMesh(core_axis_name="c", subcore_axis_name="s")

    @functools.partial(
        pl.kernel,
        out_type=jax.ShapeDtypeStruct((e, d), jnp.float32),
        mesh=mesh,
        scratch_types=[
            pltpu.VMEM((_CH,), jnp.int32),
            pltpu.VMEM((_CH,), jnp.int32),
            pltpu.VMEM((_CH, d), jnp.float32),
            pltpu.VMEM((_CH, d), jnp.float32),
            pltpu.SemaphoreType.DMA,
            pltpu.SemaphoreType.DMA,
        ],
    )
    def gk(table_hbm, idx_hbm, out_hbm, idx0, idx1, rows0, rows1, gsem, osem):
        wid = lax.axis_index("s") * _NC + lax.axis_index("c")
        base = wid * per_w

        def body(t2, carry):
            for b, (iv, rv) in enumerate(((idx0, rows0), (idx1, rows1))):
                off = base + (t2 * 2 + b) * _CH

                @pl.when(t2 >= 1)
                def _(rv=rv, off=off):
                    pltpu.make_async_copy(
                        rv, out_hbm.at[pl.ds(off, _CH)], osem).wait()

                pltpu.sync_copy(idx_hbm.at[pl.ds(off, _CH)], iv)
                pltpu.async_copy(table_hbm.at[iv], rv, gsem).wait()
                pltpu.async_copy(rv, out_hbm.at[pl.ds(off, _CH)], osem)
            return carry

        lax.fori_loop(0, n_ch // 2, body, 0)
        pltpu.make_async_copy(rows0, out_hbm.at[pl.ds(base, _CH)],
                              osem).wait()
        pltpu.make_async_copy(rows1, out_hbm.at[pl.ds(base, _CH)],
                              osem).wait()

    return gk(table, idx)


_RMLP = 512
_E1 = K * N


def _ag_body(p_ref, wd_ref, wb_ref, b1_ref, a_ref, g_ref):
    p = p_ref[:]
    a_ref[:] = jnp.dot(p, wd_ref[:], preferred_element_type=jnp.float32) + b1_ref[:]
    g_ref[:] = jnp.dot(p, wb_ref[:], preferred_element_type=jnp.float32)


def _ag(pos, wd, wb, b1):
    return pl.pallas_call(
        _ag_body,
        out_shape=(
            jax.ShapeDtypeStruct((N, 64), jnp.float32),
            jax.ShapeDtypeStruct((N, 128), jnp.float32),
        ),
    )(pos, wd, jnp.concatenate([wb, jnp.zeros((3, 64), jnp.float32)], axis=1),
      b1.reshape(1, 64))


def _h0stats_body(g_ref, a_ref, s_ref):
    @pl.when(pl.program_id(0) == 0)
    def _():
        s_ref[:] = jnp.zeros_like(s_ref)

    h0 = g_ref[:, :, 0:64] + a_ref[:][None]
    t = jnp.sum(h0, axis=0)
    u = jnp.sum(h0 * h0, axis=0)
    s_ref[0:1, :] += jnp.sum(t, axis=0, keepdims=True)
    s_ref[1:2, :] += jnp.sum(u, axis=0, keepdims=True)


def _h0stats(g3, a):
    return pl.pallas_call(
        _h0stats_body,
        grid=(N // _RMLP,),
        in_specs=[
            pl.BlockSpec((K, _RMLP, 128), lambda i: (0, i, 0)),
            pl.BlockSpec((_RMLP, 64), lambda i: (i, 0)),
        ],
        out_specs=pl.BlockSpec((8, 64), lambda i: (0, 0)),
        out_shape=jax.ShapeDtypeStruct((8, 64), jnp.float32),
    )(g3, a)


def _bn_coefs(st_ref, gamma_ref, beta_ref, count):
    mu = st_ref[0:1, :] * (1.0 / count)
    var = st_ref[1:2, :] * (1.0 / count) - mu * mu
    scale = gamma_ref[:] / jnp.sqrt(var + 1e-5)
    shift = beta_ref[:] - mu * scale
    return scale, shift


def _p2_body(g_ref, a_ref, st_ref, ga_ref, be_ref, w_ref, b_ref, h1_ref, s2_ref):
    @pl.when(pl.program_id(0) == 0)
    def _():
        s2_ref[:] = jnp.zeros_like(s2_ref)

    scale, shift = _bn_coefs(st_ref, ga_ref, be_ref, _E1)
    h0 = g_ref[:, :, 0:64] + a_ref[:][None]
    y = jnp.maximum(h0 * scale[None] + shift[None], 0.0)
    y2 = y.reshape(K * _RMLP, 64)
    h1 = jnp.dot(y2, w_ref[:], preferred_element_type=jnp.float32) + b_ref[:]
    h1_ref[:] = h1.reshape(K, _RMLP, 64)
    s2_ref[0:1, :] += jnp.sum(h1, axis=0, keepdims=True)
    s2_ref[1:2, :] += jnp.sum(h1 * h1, axis=0, keepdims=True)


def _p2(g3, a, st, gamma, beta, w2, b2):
    return pl.pallas_call(
        _p2_body,
        grid=(N // _RMLP,),
        in_specs=[
            pl.BlockSpec((K, _RMLP, 128), lambda i: (0, i, 0)),
            pl.BlockSpec((_RMLP, 64), lambda i: (i, 0)),
            pl.BlockSpec((8, 64), lambda i: (0, 0)),
            pl.BlockSpec((1, 64), lambda i: (0, 0)),
            pl.BlockSpec((1, 64), lambda i: (0, 0)),
            pl.BlockSpec((64, 64), lambda i: (0, 0)),
            pl.BlockSpec((1, 64), lambda i: (0, 0)),
        ],
        out_specs=(
            pl.BlockSpec((K, _RMLP, 64), lambda i: (0, i, 0)),
            pl.BlockSpec((8, 64), lambda i: (0, 0)),
        ),
        out_shape=(
            jax.ShapeDtypeStruct((K, N, 64), jnp.float32),
            jax.ShapeDtypeStruct((8, 64), jnp.float32),
        ),
    )(g3, a, st, gamma.reshape(1, 64), beta.reshape(1, 64), w2, b2.reshape(1, 64))


def _p3_body(h1_ref, st_ref, ga_ref, be_ref, w_ref, b_ref, x1_ref):
    scale, shift = _bn_coefs(st_ref, ga_ref, be_ref, _E1)
    y = jnp.maximum(h1_ref[:] * scale[None] + shift[None], 0.0)
    y2 = y.reshape(K * _RMLP, 64)
    h2 = jnp.dot(y2, w_ref[:], preferred_element_type=jnp.float32) + b_ref[:]
    x1_ref[:] = jnp.max(h2.reshape(K, _RMLP, 64), axis=0)


def _p3(h1, st, gamma, beta, w3, b3):
    return pl.pallas_call(
        _p3_body,
        grid=(N // _RMLP,),
        in_specs=[
            pl.BlockSpec((K, _RMLP, 64), lambda i: (0, i, 0)),
            pl.BlockSpec((8, 64), lambda i: (0, 0)),
            pl.BlockSpec((1, 64), lambda i: (0, 0)),
            pl.BlockSpec((1, 64), lambda i: (0, 0)),
            pl.BlockSpec((64, 64), lambda i: (0, 0)),
            pl.BlockSpec((1, 64), lambda i: (0, 0)),
        ],
        out_specs=pl.BlockSpec((_RMLP, 64), lambda i: (i, 0)),
        out_shape=jax.ShapeDtypeStruct((N, 64), jnp.float32),
    )(h1, st, gamma.reshape(1, 64), beta.reshape(1, 64), w3, b3.reshape(1, 64))


def _yz_body(x_ref, wz_ref, bz_ref, wy_ref, z_ref, y_ref):
    x = x_ref[:]
    z_ref[:] = jnp.dot(x, wz_ref[:], preferred_element_type=jnp.float32) + bz_ref[:]
    y_ref[:] = jnp.dot(x, wy_ref[:], preferred_element_type=jnp.float32)


def _yz(x1, wz, bz, wy):
    return pl.pallas_call(
        _yz_body,
        out_shape=(
            jax.ShapeDtypeStruct((N, 128), jnp.float32),
            jax.ShapeDtypeStruct((N, 128), jnp.float32),
        ),
    )(x1, wz, bz.reshape(1, 128), wy)


def _aggmax_body(y_ref, z_ref, x2_ref):
    x2_ref[:] = z_ref[:] + jnp.max(y_ref[:], axis=0)


def _aggmax(y3, z):
    return pl.pallas_call(
        _aggmax_body,
        grid=(N // _RMLP,),
        in_specs=[
            pl.BlockSpec((K, _RMLP, 128), lambda i: (0, i, 0)),
            pl.BlockSpec((_RMLP, 128), lambda i: (i, 0)),
        ],
        out_specs=pl.BlockSpec((_RMLP, 128), lambda i: (i, 0)),
        out_shape=jax.ShapeDtypeStruct((N, 128), jnp.float32),
    )(y3, z)


def _pool_body(x1_ref, x2_ref, bt_ref, wl_ref, bl_ref, p_ref):
    @pl.when(pl.program_id(0) == 0)
    def _():
        p_ref[:] = jnp.full_like(p_ref, -jnp.inf)

    xc = jnp.concatenate([x1_ref[:], x2_ref[:]], axis=1)
    out = jnp.dot(xc, wl_ref[:], preferred_element_type=jnp.float32) + bl_ref[:]
    bt = bt_ref[:]
    for b in range(B):
        mb = jnp.max(jnp.where(bt == b, out, -jnp.inf), axis=0, keepdims=True)
        p_ref[pl.ds(b, 1), :] = jnp.maximum(p_ref[pl.ds(b, 1), :], mb)


def _pool(x1, x2, bc, wl, bl):
    return pl.pallas_call(
        _pool_body,
        grid=(N // _RMLP,),
        in_specs=[
            pl.BlockSpec((_RMLP, 64), lambda i: (i, 0)),
            pl.BlockSpec((_RMLP, 128), lambda i: (i, 0)),
            pl.BlockSpec((_RMLP, 1), lambda i: (i, 0)),
            pl.BlockSpec((192, 1024), lambda i: (0, 0)),
            pl.BlockSpec((1, 1024), lambda i: (0, 0)),
        ],
        out_specs=pl.BlockSpec((B, 1024), lambda i: (0, 0)),
        out_shape=jax.ShapeDtypeStruct((B, 1024), jnp.float32),
    )(x1, x2, bc, wl, bl.reshape(1, 1024))


def _head_body(p_ref, w1_ref, b1_ref, w2_ref, b2_ref, w3_ref, b3_ref, o_ref):
    h = jnp.maximum(
        jnp.dot(p_ref[:], w1_ref[:], preferred_element_type=jnp.float32) + b1_ref[:], 0.0)
    h = jnp.maximum(
        jnp.dot(h, w2_ref[:], preferred_element_type=jnp.float32) + b2_ref[:], 0.0)
    h = jnp.dot(h, w3_ref[:], preferred_element_type=jnp.float32) + b3_ref[:]
    m = jnp.max(h, axis=1, keepdims=True)
    s = h - m
    o_ref[:] = s - jnp.log(jnp.sum(jnp.exp(s), axis=1, keepdims=True))


def _head(p, wm1, bm1, wm2, bm2, wm3, bm3):
    return pl.pallas_call(
        _head_body,
        out_shape=jax.ShapeDtypeStruct((B, 40), jnp.float32),
    )(p, wm1, bm1.reshape(1, 512), wm2, bm2.reshape(1, 256), wm3, bm3.reshape(1, 40))


def kernel(pos, batch, W1, b1, g1, be1, W2, b2, g2, be2, W3, b3, Wc2, bc2,
           Wl, bl, Wm1, bm1, Wm2, bm2, Wm3, bm3):
    batch = batch.astype(jnp.int32)
    bc = batch.reshape(N, 1)
    br = batch.reshape(1, N)
    c0s, nchs = _seg_ranges(batch)

    xc1, xr1 = _ext(pos, 8)
    idx1 = _knn(xc1, xr1, bc, br, c0s, nchs)
    a1, g1t = _ag(pos, W1[:3] - W1[3:], W1[3:], b1)
    gat = _sc_gather(g1t, idx1.reshape(K * N), 128)
    g3 = gat.reshape(K, N, 128)
    st1 = _h0stats(g3, a1)
    h1, st2 = _p2(g3, a1, st1, g1, be1, W2, b2)
    x1 = _p3(h1, st2, g2, be2, W3, b3)

    xc2, xr2 = _ext(x1, 72)
    idx2 = _knn(xc2, xr2, bc, br, c0s, nchs)
    z, y = _yz(x1, Wc2[:64] - Wc2[64:], bc2, Wc2[64:])
    yg = _sc_gather(y, idx2.reshape(K * N), 128)
    x2 = _aggmax(yg.reshape(K, N, 128), z)

    pooled = _pool(x1, x2, bc, Wl, bl)
    return _head(pooled, Wm1, bm1, Wm2, bm2, Wm3, bm3)


# speedup vs baseline: 18.9425x; 1.0108x over previous
import functools

import jax
import jax.numpy as jnp
from jax import lax
from jax.experimental import pallas as pl
from jax.experimental.pallas import tpu as pltpu
from jax.experimental.pallas import tpu_sc as plsc

N = 8192
B = 8
K = 20
BIG = 1e30
BIG2 = 2e30


_RKNN = 256
_CW = 512
_IBIG = 2**30


def _knn_body(c0_ref, nch_ref, x_ref, xr_ref, d2c_ref, bc_ref, br_ref,
              idx_ref, dist_ref):
    i = pl.program_id(0)
    c0 = c0_ref[i]
    nch = nch_ref[i]
    br = br_ref[:]
    xr = xr_ref[:]
    dx = xr.shape[1]
    d2r = lax.dot_general(
        jnp.ones((1, dx), jnp.float32), xr * xr,
        dimension_numbers=(((1,), (1,)), ((), ())),
        preferred_element_type=jnp.float32,
    )
    rowi0 = lax.broadcasted_iota(jnp.int32, (_CW, _RKNN), 0)
    minf = jnp.full((1, _RKNN), jnp.inf, jnp.float32)
    ibig = jnp.full((1, _RKNN), _IBIG, jnp.int32)

    def _scan_chunk(d, off, m, ji):
        lm = jnp.min(d, axis=0, keepdims=True)
        la = jnp.min(jnp.where(d == lm, rowi0, jnp.int32(_IBIG)),
                     axis=0, keepdims=True) + off
        better = lm < m
        return jnp.minimum(m, lm), jnp.where(better, la, ji)

    def fill(t, mj):
        m, ji = mj
        start = c0 + t * _CW
        mm = lax.dot_general(
            x_ref[pl.ds(start, _CW), :], xr,
            dimension_numbers=(((1,), (1,)), ((), ())),
            preferred_element_type=jnp.float32,
        )
        d = d2c_ref[pl.ds(start, _CW), :] + d2r - 2.0 * mm
        d = jnp.where(bc_ref[pl.ds(start, _CW), :] != br, BIG, d)
        dist_ref[pl.ds(t * _CW, _CW), :] = d
        return _scan_chunk(d, start, m, ji)

    m, ji = lax.fori_loop(0, nch, fill, (minf, ibig))

    picks = [ji]
    for _ in range(1, K):
        prev = ji

        def ext(t, mj, prev=prev):
            m2, ji2 = mj
            off = c0 + t * _CW
            d = dist_ref[pl.ds(t * _CW, _CW), :]
            d = jnp.where(rowi0 == (prev - off), BIG2, d)
            dist_ref[pl.ds(t * _CW, _CW), :] = d
            return _scan_chunk(d, off, m2, ji2)

        m, ji = lax.fori_loop(0, nch, ext, (minf, ibig))
        picks.append(ji)
    idx_ref[:] = jnp.concatenate(picks, axis=0)


def _knn(x, d2c, bc, br, c0s, nchs):
    dx = x.shape[1]
    grid_spec = pltpu.PrefetchScalarGridSpec(
        num_scalar_prefetch=2,
        grid=(N // _RKNN,),
        in_specs=[
            pl.BlockSpec((N, dx), lambda i, *_: (0, 0)),
            pl.BlockSpec((_RKNN, dx), lambda i, *_: (i, 0)),
            pl.BlockSpec((N, 1), lambda i, *_: (0, 0)),
            pl.BlockSpec((N, 1), lambda i, *_: (0, 0)),
            pl.BlockSpec((1, _RKNN), lambda i, *_: (0, i)),
        ],
        out_specs=pl.BlockSpec((K, _RKNN), lambda i, *_: (0, i)),
        scratch_shapes=[pltpu.VMEM((N, _RKNN), jnp.float32)],
    )
    return pl.pallas_call(
        _knn_body,
        grid_spec=grid_spec,
        out_shape=jax.ShapeDtypeStruct((K, N), jnp.int32),
    )(c0s, nchs, x, x, d2c, bc, br)


def _seg_ranges(batch):
    ar = jnp.arange(B, dtype=batch.dtype)
    seg_start = jnp.searchsorted(batch, ar).astype(jnp.int32)
    seg_end = jnp.searchsorted(batch, ar, side="right").astype(jnp.int32)
    sizes = seg_end - seg_start
    degenerate = jnp.any((sizes > 0) & (sizes < K))
    b2 = batch.reshape(N // _RKNN, _RKNN)
    first_b = b2[:, 0]
    last_b = b2[:, -1]
    c0 = (seg_start[first_b] // _CW) * _CW
    nch = (seg_end[last_b] - c0 + _CW - 1) // _CW
    c0 = jnp.where(degenerate, 0, c0).astype(jnp.int32)
    nch = jnp.where(degenerate, N // _CW, nch).astype(jnp.int32)
    return c0, nch


_NC = 2
_NS = 16
_NW = _NC * _NS
_CH = 256


def _sc_gather(table, idx, d):
    e = idx.shape[0]
    per_w = e // _NW
    n_ch = per_w // _CH
    mesh = plsc.VectorSubcoreMesh(core_axis_name="c", subcore_axis_name="s")

    @functools.partial(
        pl.kernel,
        out_type=jax.ShapeDtypeStruct((e, d), jnp.float32),
        mesh=mesh,
        scratch_types=[
            pltpu.VMEM((_CH,), jnp.int32),
            pltpu.VMEM((_CH,), jnp.int32),
            pltpu.VMEM((_CH, d), jnp.float32),
            pltpu.VMEM((_CH, d), jnp.float32),
            pltpu.SemaphoreType.DMA,
            pltpu.SemaphoreType.DMA,
        ],
    )
    def gk(table_hbm, idx_hbm, out_hbm, idx0, idx1, rows0, rows1, gsem, osem):
        wid = lax.axis_index("s") * _NC + lax.axis_index("c")
        base = wid * per_w

        def body(t2, carry):
            for b, (iv, rv) in enumerate(((idx0, rows0), (idx1, rows1))):
                off = base + (t2 * 2 + b) * _CH

                @pl.when(t2 >= 1)
                def _(rv=rv, off=off):
                    pltpu.make_async_copy(
                        rv, out_hbm.at[pl.ds(off, _CH)], osem).wait()

                pltpu.sync_copy(idx_hbm.at[pl.ds(off, _CH)], iv)
                pltpu.async_copy(table_hbm.at[iv], rv, gsem).wait()
                pltpu.async_copy(rv, out_hbm.at[pl.ds(off, _CH)], osem)
            return carry

        lax.fori_loop(0, n_ch // 2, body, 0)
        pltpu.make_async_copy(rows0, out_hbm.at[pl.ds(base, _CH)],
                              osem).wait()
        pltpu.make_async_copy(rows1, out_hbm.at[pl.ds(base, _CH)],
                              osem).wait()

    return gk(table, idx)


_RMLP = 512
_E1 = K * N


def _ag_body(p_ref, wd_ref, wb_ref, b1_ref, a_ref, g_ref, d2_ref):
    p = p_ref[:]
    a_ref[:] = jnp.dot(p, wd_ref[:], preferred_element_type=jnp.float32) + b1_ref[:]
    g_ref[:] = jnp.dot(p, wb_ref[:], preferred_element_type=jnp.float32)
    d2_ref[:] = jnp.sum(p * p, axis=1, keepdims=True)


def _ag(pos, wd, wb, b1):
    return pl.pallas_call(
        _ag_body,
        out_shape=(
            jax.ShapeDtypeStruct((N, 64), jnp.float32),
            jax.ShapeDtypeStruct((N, 128), jnp.float32),
            jax.ShapeDtypeStruct((N, 1), jnp.float32),
        ),
    )(pos, wd, jnp.concatenate([wb, jnp.zeros((3, 64), jnp.float32)], axis=1),
      b1.reshape(1, 64))


def _bn_coefs(st_ref, gamma_ref, beta_ref, count):
    mu = st_ref[0:1, :] * (1.0 / count)
    var = st_ref[1:2, :] * (1.0 / count) - mu * mu
    scale = gamma_ref[:] / jnp.sqrt(var + 1e-5)
    shift = beta_ref[:] - mu * scale
    return scale, shift


def _p2_body(g_ref, a_ref, ga_ref, be_ref, w_ref, b_ref, h1_ref, s_ref,
             s2_ref):
    p = pl.program_id(0)
    i = pl.program_id(1)
    h0 = g_ref[:, :, 0:64] + a_ref[:][None]

    @pl.when(p == 0)
    def _():
        @pl.when(i == 0)
        def _():
            s_ref[:] = jnp.zeros_like(s_ref)

        t = jnp.sum(h0, axis=0)
        u = jnp.sum(h0 * h0, axis=0)
        s_ref[0:1, :] += jnp.sum(t, axis=0, keepdims=True)
        s_ref[1:2, :] += jnp.sum(u, axis=0, keepdims=True)

    @pl.when(p == 1)
    def _():
        @pl.when(i == 0)
        def _():
            s2_ref[:] = jnp.zeros_like(s2_ref)

        scale, shift = _bn_coefs(s_ref, ga_ref, be_ref, _E1)
        y = jnp.maximum(h0 * scale[None] + shift[None], 0.0)
        y2 = y.reshape(K * _RMLP, 64)
        h1 = jnp.dot(y2, w_ref[:], preferred_element_type=jnp.float32) + b_ref[:]
        h1_ref[:] = h1.reshape(K, _RMLP, 64)
        s2_ref[0:1, :] += jnp.sum(h1, axis=0, keepdims=True)
        s2_ref[1:2, :] += jnp.sum(h1 * h1, axis=0, keepdims=True)


def _p2(g3, a, gamma, beta, w2, b2):
    return pl.pallas_call(
        _p2_body,
        grid=(2, N // _RMLP),
        in_specs=[
            pl.BlockSpec((K, _RMLP, 128), lambda p, i: (0, i, 0)),
            pl.BlockSpec((_RMLP, 64), lambda p, i: (i, 0)),
            pl.BlockSpec((1, 64), lambda p, i: (0, 0)),
            pl.BlockSpec((1, 64), lambda p, i: (0, 0)),
            pl.BlockSpec((64, 64), lambda p, i: (0, 0)),
            pl.BlockSpec((1, 64), lambda p, i: (0, 0)),
        ],
        out_specs=(
            pl.BlockSpec((K, _RMLP, 64), lambda p, i: (0, i * p, 0)),
            pl.BlockSpec((8, 64), lambda p, i: (0, 0)),
            pl.BlockSpec((8, 64), lambda p, i: (0, 0)),
        ),
        out_shape=(
            jax.ShapeDtypeStruct((K, N, 64), jnp.float32),
            jax.ShapeDtypeStruct((8, 64), jnp.float32),
            jax.ShapeDtypeStruct((8, 64), jnp.float32),
        ),
    )(g3, a, gamma.reshape(1, 64), beta.reshape(1, 64), w2, b2.reshape(1, 64))


def _p3_body(h1_ref, st_ref, ga_ref, be_ref, w_ref, b_ref, wz_ref, bz_ref,
             wy_ref, x1_ref, z_ref, y_ref, d2_ref):
    scale, shift = _bn_coefs(st_ref, ga_ref, be_ref, _E1)
    yact = jnp.maximum(h1_ref[:] * scale[None] + shift[None], 0.0)
    y2 = yact.reshape(K * _RMLP, 64)
    h2 = jnp.dot(y2, w_ref[:], preferred_element_type=jnp.float32) + b_ref[:]
    x1 = jnp.max(h2.reshape(K, _RMLP, 64), axis=0)
    x1_ref[:] = x1
    z_ref[:] = jnp.dot(x1, wz_ref[:], preferred_element_type=jnp.float32) + bz_ref[:]
    y_ref[:] = jnp.dot(x1, wy_ref[:], preferred_element_type=jnp.float32)
    d2_ref[:] = jnp.sum(x1 * x1, axis=1, keepdims=True)


def _p3(h1, st, gamma, beta, w3, b3, wz, bz, wy):
    return pl.pallas_call(
        _p3_body,
        grid=(N // _RMLP,),
        in_specs=[
            pl.BlockSpec((K, _RMLP, 64), lambda i: (0, i, 0)),
            pl.BlockSpec((8, 64), lambda i: (0, 0)),
            pl.BlockSpec((1, 64), lambda i: (0, 0)),
            pl.BlockSpec((1, 64), lambda i: (0, 0)),
            pl.BlockSpec((64, 64), lambda i: (0, 0)),
            pl.BlockSpec((1, 64), lambda i: (0, 0)),
            pl.BlockSpec((64, 128), lambda i: (0, 0)),
            pl.BlockSpec((1, 128), lambda i: (0, 0)),
            pl.BlockSpec((64, 128), lambda i: (0, 0)),
        ],
        out_specs=(
            pl.BlockSpec((_RMLP, 64), lambda i: (i, 0)),
            pl.BlockSpec((_RMLP, 128), lambda i: (i, 0)),
            pl.BlockSpec((_RMLP, 128), lambda i: (i, 0)),
            pl.BlockSpec((_RMLP, 1), lambda i: (i, 0)),
        ),
        out_shape=(
            jax.ShapeDtypeStruct((N, 64), jnp.float32),
            jax.ShapeDtypeStruct((N, 128), jnp.float32),
            jax.ShapeDtypeStruct((N, 128), jnp.float32),
            jax.ShapeDtypeStruct((N, 1), jnp.float32),
        ),
    )(h1, st, gamma.reshape(1, 64), beta.reshape(1, 64), w3, b3.reshape(1, 64),
      wz, bz.reshape(1, 128), wy)


_NPB = N // _RMLP


def _pool_body(x1_ref, yg_ref, z_ref, bt_ref, wl_ref, bl_ref,
               wm1_ref, bm1_ref, wm2_ref, bm2_ref, wm3_ref, bm3_ref,
               o_ref, p_ref):
    i = pl.program_id(0)

    @pl.when(i == 0)
    def _():
        p_ref[:] = jnp.full_like(p_ref, -jnp.inf)

    @pl.when(i < _NPB)
    def _():
        x2 = z_ref[:] + jnp.max(yg_ref[:], axis=0)
        xc = jnp.concatenate([x1_ref[:], x2], axis=1)
        out = jnp.dot(xc, wl_ref[:], preferred_element_type=jnp.float32) + bl_ref[:]
        bt = bt_ref[:]
        for b in range(B):
            mb = jnp.max(jnp.where(bt == b, out, -jnp.inf), axis=0,
                         keepdims=True)
            p_ref[pl.ds(b, 1), :] = jnp.maximum(p_ref[pl.ds(b, 1), :], mb)

    @pl.when(i == _NPB)
    def _():
        h = jnp.maximum(
            jnp.dot(p_ref[:], wm1_ref[:],
                    preferred_element_type=jnp.float32) + bm1_ref[:], 0.0)
        h = jnp.maximum(
            jnp.dot(h, wm2_ref[:],
                    preferred_element_type=jnp.float32) + bm2_ref[:], 0.0)
        h = jnp.dot(h, wm3_ref[:], preferred_element_type=jnp.float32) + bm3_ref[:]
        m = jnp.max(h, axis=1, keepdims=True)
        s = h - m
        o_ref[:] = s - jnp.log(jnp.sum(jnp.exp(s), axis=1, keepdims=True))


def _pool_head(x1, yg3, z, bc, wl, bl, wm1, bm1, wm2, bm2, wm3, bm3):
    blk = lambda i: jnp.minimum(i, _NPB - 1)
    return pl.pallas_call(
        _pool_body,
        grid=(_NPB + 1,),
        in_specs=[
            pl.BlockSpec((_RMLP, 64), lambda i: (blk(i), 0)),
            pl.BlockSpec((K, _RMLP, 128), lambda i: (0, blk(i), 0)),
            pl.BlockSpec((_RMLP, 128), lambda i: (blk(i), 0)),
            pl.BlockSpec((_RMLP, 1), lambda i: (blk(i), 0)),
            pl.BlockSpec((192, 1024), lambda i: (0, 0)),
            pl.BlockSpec((1, 1024), lambda i: (0, 0)),
            pl.BlockSpec((1024, 512), lambda i: (0, 0)),
            pl.BlockSpec((1, 512), lambda i: (0, 0)),
            pl.BlockSpec((512, 256), lambda i: (0, 0)),
            pl.BlockSpec((1, 256), lambda i: (0, 0)),
            pl.BlockSpec((256, 40), lambda i: (0, 0)),
            pl.BlockSpec((1, 40), lambda i: (0, 0)),
        ],
        out_specs=pl.BlockSpec((B, 40), lambda i: (0, 0)),
        out_shape=jax.ShapeDtypeStruct((B, 40), jnp.float32),
        scratch_shapes=[pltpu.VMEM((B, 1024), jnp.float32)],
    )(x1, yg3, z, bc, wl, bl.reshape(1, 1024), wm1, bm1.reshape(1, 512),
      wm2, bm2.reshape(1, 256), wm3, bm3.reshape(1, 40))


def kernel(pos, batch, W1, b1, g1, be1, W2, b2, g2, be2, W3, b3, Wc2, bc2,
           Wl, bl, Wm1, bm1, Wm2, bm2, Wm3, bm3):
    batch = batch.astype(jnp.int32)
    bc = batch.reshape(N, 1)
    br = batch.reshape(1, N)
    c0s, nchs = _seg_ranges(batch)

    a1, g1t, d2p = _ag(pos, W1[:3] - W1[3:], W1[3:], b1)
    idx1 = _knn(pos, d2p, bc, br, c0s, nchs)
    gat = _sc_gather(g1t, idx1.reshape(K * N), 128)
    g3 = gat.reshape(K, N, 128)
    h1, st1, st2 = _p2(g3, a1, g1, be1, W2, b2)
    x1, z, y, d2x = _p3(h1, st2, g2, be2, W3, b3,
                        Wc2[:64] - Wc2[64:], bc2, Wc2[64:])

    idx2 = _knn(x1, d2x, bc, br, c0s, nchs)
    yg = _sc_gather(y, idx2.reshape(K * N), 128)

    return _pool_head(x1, yg.reshape(K, N, 128), z, bc, Wl, bl,
                      Wm1, bm1, Wm2, bm2, Wm3, bm3)


# speedup vs baseline: 20.4427x; 1.0792x over previous
import functools

import jax
import jax.numpy as jnp
from jax import lax
from jax.experimental import pallas as pl
from jax.experimental.pallas import tpu as pltpu
from jax.experimental.pallas import tpu_sc as plsc

N = 8192
B = 8
K = 20
BIG = 1e30
BIG2 = 2e30


_RKNN = 256
_CW = 512
_IBIG = 2**30


def _knn_body(c0_ref, nch_ref, x_ref, xr_ref, d2c_ref, bc_ref, br_ref,
              idx_ref, dist_ref):
    i = pl.program_id(0)
    c0 = c0_ref[i]
    nch = nch_ref[i]
    br = br_ref[:]
    xr = xr_ref[:]
    dx = xr.shape[1]
    d2r = lax.dot_general(
        jnp.ones((1, dx), jnp.float32), xr * xr,
        dimension_numbers=(((1,), (1,)), ((), ())),
        preferred_element_type=jnp.float32,
    )
    rowi0 = lax.broadcasted_iota(jnp.int32, (_CW, _RKNN), 0)
    minf = jnp.full((1, _RKNN), jnp.inf, jnp.float32)
    ibig = jnp.full((1, _RKNN), _IBIG, jnp.int32)

    def _scan_chunk(d, off, m, ji):
        lm = jnp.min(d, axis=0, keepdims=True)
        la = jnp.min(jnp.where(d == lm, rowi0, jnp.int32(_IBIG)),
                     axis=0, keepdims=True) + off
        better = lm < m
        return jnp.minimum(m, lm), jnp.where(better, la, ji)

    def fill(t, mj):
        m, ji = mj
        start = jnp.minimum(c0 + t * _CW, N - _CW)
        mm = lax.dot_general(
            x_ref[pl.ds(start, _CW), :], xr,
            dimension_numbers=(((1,), (1,)), ((), ())),
            preferred_element_type=jnp.float32,
        )
        d = d2c_ref[pl.ds(start, _CW), :] + d2r - 2.0 * mm
        d = jnp.where(bc_ref[pl.ds(start, _CW), :] != br, BIG, d)
        dist_ref[pl.ds(t * _CW, _CW), :] = d
        return _scan_chunk(d, start, m, ji)

    m, ji = lax.fori_loop(0, nch, fill, (minf, ibig))

    picks = [ji]
    for step in range(1, K):
        prev = ji
        last = step == K - 1

        def ext(t, mj, prev=prev, last=last):
            m2, ji2 = mj
            off = jnp.minimum(c0 + t * _CW, N - _CW)
            d = dist_ref[pl.ds(t * _CW, _CW), :]
            d = jnp.where(rowi0 == (prev - off), BIG2, d)
            if not last:
                dist_ref[pl.ds(t * _CW, _CW), :] = d
            return _scan_chunk(d, off, m2, ji2)

        m, ji = lax.fori_loop(0, nch, ext, (minf, ibig))
        picks.append(ji)
    idx_ref[:] = jnp.concatenate(picks, axis=0)


def _knn(x, d2c, bc, br, c0s, nchs):
    dx = x.shape[1]
    grid_spec = pltpu.PrefetchScalarGridSpec(
        num_scalar_prefetch=2,
        grid=(N // _RKNN,),
        in_specs=[
            pl.BlockSpec((N, dx), lambda i, *_: (0, 0)),
            pl.BlockSpec((_RKNN, dx), lambda i, *_: (i, 0)),
            pl.BlockSpec((N, 1), lambda i, *_: (0, 0)),
            pl.BlockSpec((N, 1), lambda i, *_: (0, 0)),
            pl.BlockSpec((1, _RKNN), lambda i, *_: (0, i)),
        ],
        out_specs=pl.BlockSpec((K, _RKNN), lambda i, *_: (0, i)),
        scratch_shapes=[pltpu.VMEM((N, _RKNN), jnp.float32)],
    )
    return pl.pallas_call(
        _knn_body,
        grid_spec=grid_spec,
        out_shape=jax.ShapeDtypeStruct((K, N), jnp.int32),
    )(c0s, nchs, x, x, d2c, bc, br)


def _seg_ranges(batch):
    ar = jnp.arange(B, dtype=batch.dtype)
    seg_start = jnp.searchsorted(batch, ar).astype(jnp.int32)
    seg_end = jnp.searchsorted(batch, ar, side="right").astype(jnp.int32)
    sizes = seg_end - seg_start
    degenerate = jnp.any((sizes > 0) & (sizes < K))
    b2 = batch.reshape(N // _RKNN, _RKNN)
    first_b = b2[:, 0]
    last_b = b2[:, -1]
    c0 = jnp.minimum((seg_start[first_b] // 8) * 8, N - _CW)
    nch = (seg_end[last_b] - c0 + _CW - 1) // _CW
    c0 = jnp.where(degenerate, 0, c0).astype(jnp.int32)
    nch = jnp.where(degenerate, N // _CW, nch).astype(jnp.int32)
    return c0, nch


_NC = 2
_NS = 16
_NW = _NC * _NS
_CH = 256


def _sc_gather(table, idx, d):
    e = idx.shape[0]
    per_w = e // _NW
    n_ch = per_w // _CH
    mesh = plsc.VectorSubcoreMesh(core_axis_name="c", subcore_axis_name="s")

    @functools.partial(
        pl.kernel,
        out_type=jax.ShapeDtypeStruct((e, d), jnp.float32),
        mesh=mesh,
        scratch_types=[
            pltpu.VMEM((_CH,), jnp.int32),
            pltpu.VMEM((_CH,), jnp.int32),
            pltpu.VMEM((_CH, d), jnp.float32),
            pltpu.VMEM((_CH, d), jnp.float32),
            pltpu.SemaphoreType.DMA,
            pltpu.SemaphoreType.DMA,
        ],
    )
    def gk(table_hbm, idx_hbm, out_hbm, idx0, idx1, rows0, rows1, gsem, osem):
        wid = lax.axis_index("s") * _NC + lax.axis_index("c")
        base = wid * per_w

        def body(t2, carry):
            for b, (iv, rv) in enumerate(((idx0, rows0), (idx1, rows1))):
                off = base + (t2 * 2 + b) * _CH

                @pl.when(t2 >= 1)
                def _(rv=rv, off=off):
                    pltpu.make_async_copy(
                        rv, out_hbm.at[pl.ds(off, _CH)], osem).wait()

                pltpu.sync_copy(idx_hbm.at[pl.ds(off, _CH)], iv)
                pltpu.async_copy(table_hbm.at[iv], rv, gsem).wait()
                pltpu.async_copy(rv, out_hbm.at[pl.ds(off, _CH)], osem)
            return carry

        lax.fori_loop(0, n_ch // 2, body, 0)
        pltpu.make_async_copy(rows0, out_hbm.at[pl.ds(base, _CH)],
                              osem).wait()
        pltpu.make_async_copy(rows1, out_hbm.at[pl.ds(base, _CH)],
                              osem).wait()

    return gk(table, idx)


_RMLP = 512
_E1 = K * N


def _ag_body(p_ref, wd_ref, wb_ref, b1_ref, a_ref, g_ref, d2_ref):
    p = p_ref[:]
    a_ref[:] = jnp.dot(p, wd_ref[:], preferred_element_type=jnp.float32) + b1_ref[:]
    g_ref[:] = jnp.dot(p, wb_ref[:], preferred_element_type=jnp.float32)
    d2_ref[:] = jnp.sum(p * p, axis=1, keepdims=True)


def _ag(pos, wd, wb, b1):
    return pl.pallas_call(
        _ag_body,
        out_shape=(
            jax.ShapeDtypeStruct((N, 64), jnp.float32),
            jax.ShapeDtypeStruct((N, 128), jnp.float32),
            jax.ShapeDtypeStruct((N, 1), jnp.float32),
        ),
    )(pos, wd, jnp.concatenate([wb, jnp.zeros((3, 64), jnp.float32)], axis=1),
      b1.reshape(1, 64))


def _bn_coefs(st_ref, gamma_ref, beta_ref, count):
    mu = st_ref[0:1, :] * (1.0 / count)
    var = st_ref[1:2, :] * (1.0 / count) - mu * mu
    scale = gamma_ref[:] / jnp.sqrt(var + 1e-5)
    shift = beta_ref[:] - mu * scale
    return scale, shift


def _p2_body(g_ref, a_ref, ga_ref, be_ref, w_ref, b_ref, h1_ref, s_ref,
             s2_ref):
    p = pl.program_id(0)
    i = pl.program_id(1)
    h0 = g_ref[:, :, 0:64] + a_ref[:][None]

    @pl.when(p == 0)
    def _():
        @pl.when(i == 0)
        def _():
            s_ref[:] = jnp.zeros_like(s_ref)

        t = jnp.sum(h0, axis=0)
        u = jnp.sum(h0 * h0, axis=0)
        s_ref[0:1, :] += jnp.sum(t, axis=0, keepdims=True)
        s_ref[1:2, :] += jnp.sum(u, axis=0, keepdims=True)

    @pl.when(p == 1)
    def _():
        @pl.when(i == 0)
        def _():
            s2_ref[:] = jnp.zeros_like(s2_ref)

        scale, shift = _bn_coefs(s_ref, ga_ref, be_ref, _E1)
        y = jnp.maximum(h0 * scale[None] + shift[None], 0.0)
        y2 = y.reshape(K * _RMLP, 64)
        h1 = jnp.dot(y2, w_ref[:], preferred_element_type=jnp.float32) + b_ref[:]
        h1_ref[:] = h1.reshape(K, _RMLP, 64)
        s2_ref[0:1, :] += jnp.sum(h1, axis=0, keepdims=True)
        s2_ref[1:2, :] += jnp.sum(h1 * h1, axis=0, keepdims=True)


def _p2(g3, a, gamma, beta, w2, b2):
    return pl.pallas_call(
        _p2_body,
        grid=(2, N // _RMLP),
        in_specs=[
            pl.BlockSpec((K, _RMLP, 128), lambda p, i: (0, i, 0)),
            pl.BlockSpec((_RMLP, 64), lambda p, i: (i, 0)),
            pl.BlockSpec((1, 64), lambda p, i: (0, 0)),
            pl.BlockSpec((1, 64), lambda p, i: (0, 0)),
            pl.BlockSpec((64, 64), lambda p, i: (0, 0)),
            pl.BlockSpec((1, 64), lambda p, i: (0, 0)),
        ],
        out_specs=(
            pl.BlockSpec((K, _RMLP, 64), lambda p, i: (0, i * p, 0)),
            pl.BlockSpec((8, 64), lambda p, i: (0, 0)),
            pl.BlockSpec((8, 64), lambda p, i: (0, 0)),
        ),
        out_shape=(
            jax.ShapeDtypeStruct((K, N, 64), jnp.float32),
            jax.ShapeDtypeStruct((8, 64), jnp.float32),
            jax.ShapeDtypeStruct((8, 64), jnp.float32),
        ),
    )(g3, a, gamma.reshape(1, 64), beta.reshape(1, 64), w2, b2.reshape(1, 64))


def _p3_body(h1_ref, st_ref, ga_ref, be_ref, w_ref, b_ref, wz_ref, bz_ref,
             wy_ref, x1_ref, z_ref, y_ref, d2_ref):
    scale, shift = _bn_coefs(st_ref, ga_ref, be_ref, _E1)
    yact = jnp.maximum(h1_ref[:] * scale[None] + shift[None], 0.0)
    y2 = yact.reshape(K * _RMLP, 64)
    h2 = jnp.dot(y2, w_ref[:], preferred_element_type=jnp.float32) + b_ref[:]
    x1 = jnp.max(h2.reshape(K, _RMLP, 64), axis=0)
    x1_ref[:] = x1
    z_ref[:] = jnp.dot(x1, wz_ref[:], preferred_element_type=jnp.float32) + bz_ref[:]
    y_ref[:] = jnp.dot(x1, wy_ref[:], preferred_element_type=jnp.float32)
    d2_ref[:] = jnp.sum(x1 * x1, axis=1, keepdims=True)


def _p3(h1, st, gamma, beta, w3, b3, wz, bz, wy):
    return pl.pallas_call(
        _p3_body,
        grid=(N // _RMLP,),
        in_specs=[
            pl.BlockSpec((K, _RMLP, 64), lambda i: (0, i, 0)),
            pl.BlockSpec((8, 64), lambda i: (0, 0)),
            pl.BlockSpec((1, 64), lambda i: (0, 0)),
            pl.BlockSpec((1, 64), lambda i: (0, 0)),
            pl.BlockSpec((64, 64), lambda i: (0, 0)),
            pl.BlockSpec((1, 64), lambda i: (0, 0)),
            pl.BlockSpec((64, 128), lambda i: (0, 0)),
            pl.BlockSpec((1, 128), lambda i: (0, 0)),
            pl.BlockSpec((64, 128), lambda i: (0, 0)),
        ],
        out_specs=(
            pl.BlockSpec((_RMLP, 64), lambda i: (i, 0)),
            pl.BlockSpec((_RMLP, 128), lambda i: (i, 0)),
            pl.BlockSpec((_RMLP, 128), lambda i: (i, 0)),
            pl.BlockSpec((_RMLP, 1), lambda i: (i, 0)),
        ),
        out_shape=(
            jax.ShapeDtypeStruct((N, 64), jnp.float32),
            jax.ShapeDtypeStruct((N, 128), jnp.float32),
            jax.ShapeDtypeStruct((N, 128), jnp.float32),
            jax.ShapeDtypeStruct((N, 1), jnp.float32),
        ),
    )(h1, st, gamma.reshape(1, 64), beta.reshape(1, 64), w3, b3.reshape(1, 64),
      wz, bz.reshape(1, 128), wy)


_NPB = N // _RMLP


def _pool_body(x1_ref, yg_ref, z_ref, bt_ref, wl_ref, bl_ref,
               wm1_ref, bm1_ref, wm2_ref, bm2_ref, wm3_ref, bm3_ref,
               o_ref, p_ref):
    i = pl.program_id(0)

    @pl.when(i == 0)
    def _():
        p_ref[:] = jnp.full_like(p_ref, -jnp.inf)

    @pl.when(i < _NPB)
    def _():
        x2 = z_ref[:] + jnp.max(yg_ref[:], axis=0)
        xc = jnp.concatenate([x1_ref[:], x2], axis=1)
        out = jnp.dot(xc, wl_ref[:], preferred_element_type=jnp.float32) + bl_ref[:]
        bt = bt_ref[:]
        for b in range(B):
            mb = jnp.max(jnp.where(bt == b, out, -jnp.inf), axis=0,
                         keepdims=True)
            p_ref[pl.ds(b, 1), :] = jnp.maximum(p_ref[pl.ds(b, 1), :], mb)

    @pl.when(i == _NPB)
    def _():
        h = jnp.maximum(
            jnp.dot(p_ref[:], wm1_ref[:],
                    preferred_element_type=jnp.float32) + bm1_ref[:], 0.0)
        h = jnp.maximum(
            jnp.dot(h, wm2_ref[:],
                    preferred_element_type=jnp.float32) + bm2_ref[:], 0.0)
        h = jnp.dot(h, wm3_ref[:], preferred_element_type=jnp.float32) + bm3_ref[:]
        m = jnp.max(h, axis=1, keepdims=True)
        s = h - m
        o_ref[:] = s - jnp.log(jnp.sum(jnp.exp(s), axis=1, keepdims=True))


def _pool_head(x1, yg3, z, bc, wl, bl, wm1, bm1, wm2, bm2, wm3, bm3):
    blk = lambda i: jnp.minimum(i, _NPB - 1)
    return pl.pallas_call(
        _pool_body,
        grid=(_NPB + 1,),
        in_specs=[
            pl.BlockSpec((_RMLP, 64), lambda i: (blk(i), 0)),
            pl.BlockSpec((K, _RMLP, 128), lambda i: (0, blk(i), 0)),
            pl.BlockSpec((_RMLP, 128), lambda i: (blk(i), 0)),
            pl.BlockSpec((_RMLP, 1), lambda i: (blk(i), 0)),
            pl.BlockSpec((192, 1024), lambda i: (0, 0)),
            pl.BlockSpec((1, 1024), lambda i: (0, 0)),
            pl.BlockSpec((1024, 512), lambda i: (0, 0)),
            pl.BlockSpec((1, 512), lambda i: (0, 0)),
            pl.BlockSpec((512, 256), lambda i: (0, 0)),
            pl.BlockSpec((1, 256), lambda i: (0, 0)),
            pl.BlockSpec((256, 40), lambda i: (0, 0)),
            pl.BlockSpec((1, 40), lambda i: (0, 0)),
        ],
        out_specs=pl.BlockSpec((B, 40), lambda i: (0, 0)),
        out_shape=jax.ShapeDtypeStruct((B, 40), jnp.float32),
        scratch_shapes=[pltpu.VMEM((B, 1024), jnp.float32)],
    )(x1, yg3, z, bc, wl, bl.reshape(1, 1024), wm1, bm1.reshape(1, 512),
      wm2, bm2.reshape(1, 256), wm3, bm3.reshape(1, 40))


def kernel(pos, batch, W1, b1, g1, be1, W2, b2, g2, be2, W3, b3, Wc2, bc2,
           Wl, bl, Wm1, bm1, Wm2, bm2, Wm3, bm3):
    batch = batch.astype(jnp.int32)
    bc = batch.reshape(N, 1)
    br = batch.reshape(1, N)
    c0s, nchs = _seg_ranges(batch)

    a1, g1t, d2p = _ag(pos, W1[:3] - W1[3:], W1[3:], b1)
    idx1 = _knn(pos, d2p, bc, br, c0s, nchs)
    gat = _sc_gather(g1t, idx1.reshape(K * N), 128)
    g3 = gat.reshape(K, N, 128)
    h1, st1, st2 = _p2(g3, a1, g1, be1, W2, b2)
    x1, z, y, d2x = _p3(h1, st2, g2, be2, W3, b3,
                        Wc2[:64] - Wc2[64:], bc2, Wc2[64:])

    idx2 = _knn(x1, d2x, bc, br, c0s, nchs)
    yg = _sc_gather(y, idx2.reshape(K * N), 128)

    return _pool_head(x1, yg.reshape(K, N, 128), z, bc, Wl, bl,
                      Wm1, bm1, Wm2, bm2, Wm3, bm3)


# speedup vs baseline: 20.4480x; 1.0003x over previous
import functools

import jax
import jax.numpy as jnp
from jax import lax
from jax.experimental import pallas as pl
from jax.experimental.pallas import tpu as pltpu
from jax.experimental.pallas import tpu_sc as plsc

N = 8192
B = 8
K = 20
BIG = 1e30
BIG2 = 2e30


_RKNN = 256
_CW = 512
_IBIG = 2**30


def _knn_body(c0_ref, nch_ref, x_ref, xr_ref, d2c_ref, bc_ref, br_ref,
              idx_ref, dist_ref):
    i = pl.program_id(0)
    c0 = c0_ref[i]
    nch = nch_ref[i]
    br = br_ref[:]
    xr = xr_ref[:]
    dx = xr.shape[1]
    d2r = lax.dot_general(
        jnp.ones((1, dx), jnp.float32), xr * xr,
        dimension_numbers=(((1,), (1,)), ((), ())),
        preferred_element_type=jnp.float32,
    )
    rowi0 = lax.broadcasted_iota(jnp.int32, (_CW, _RKNN), 0)
    minf = jnp.full((1, _RKNN), jnp.inf, jnp.float32)
    ibig = jnp.full((1, _RKNN), _IBIG, jnp.int32)

    def _scan_chunk(d, off, m, ji):
        lm = jnp.min(d, axis=0, keepdims=True)
        la = jnp.min(jnp.where(d == lm, rowi0, jnp.int32(_IBIG)),
                     axis=0, keepdims=True) + off
        better = lm < m
        return jnp.minimum(m, lm), jnp.where(better, la, ji)

    def fill(t, mj):
        m, ji = mj
        start = jnp.minimum(c0 + t * _CW, N - _CW)
        mm = lax.dot_general(
            x_ref[pl.ds(start, _CW), :], xr,
            dimension_numbers=(((1,), (1,)), ((), ())),
            preferred_element_type=jnp.float32,
            precision=lax.Precision.DEFAULT,
        )
        d = d2c_ref[pl.ds(start, _CW), :] + d2r - 2.0 * mm
        d = jnp.where(bc_ref[pl.ds(start, _CW), :] != br, BIG, d)
        dist_ref[pl.ds(t * _CW, _CW), :] = d
        return _scan_chunk(d, start, m, ji)

    m, ji = lax.fori_loop(0, nch, fill, (minf, ibig))

    picks = [ji]
    for step in range(1, K):
        prev = ji
        last = step == K - 1

        def ext(t, mj, prev=prev, last=last):
            m2, ji2 = mj
            off = jnp.minimum(c0 + t * _CW, N - _CW)
            d = dist_ref[pl.ds(t * _CW, _CW), :]
            d = jnp.where(rowi0 == (prev - off), BIG2, d)
            if not last:
                dist_ref[pl.ds(t * _CW, _CW), :] = d
            return _scan_chunk(d, off, m2, ji2)

        m, ji = lax.fori_loop(0, nch, ext, (minf, ibig))
        picks.append(ji)
    idx_ref[:] = jnp.concatenate(picks, axis=0)


def _knn(x, d2c, bc, br, c0s, nchs):
    dx = x.shape[1]
    grid_spec = pltpu.PrefetchScalarGridSpec(
        num_scalar_prefetch=2,
        grid=(N // _RKNN,),
        in_specs=[
            pl.BlockSpec((N, dx), lambda i, *_: (0, 0)),
            pl.BlockSpec((_RKNN, dx), lambda i, *_: (i, 0)),
            pl.BlockSpec((N, 1), lambda i, *_: (0, 0)),
            pl.BlockSpec((N, 1), lambda i, *_: (0, 0)),
            pl.BlockSpec((1, _RKNN), lambda i, *_: (0, i)),
        ],
        out_specs=pl.BlockSpec((K, _RKNN), lambda i, *_: (0, i)),
        scratch_shapes=[pltpu.VMEM((N, _RKNN), jnp.float32)],
    )
    return pl.pallas_call(
        _knn_body,
        grid_spec=grid_spec,
        out_shape=jax.ShapeDtypeStruct((K, N), jnp.int32),
    )(c0s, nchs, x, x, d2c, bc, br)


def _seg_ranges(batch):
    ar = jnp.arange(B, dtype=batch.dtype)
    seg_start = jnp.searchsorted(batch, ar).astype(jnp.int32)
    seg_end = jnp.searchsorted(batch, ar, side="right").astype(jnp.int32)
    sizes = seg_end - seg_start
    degenerate = jnp.any((sizes > 0) & (sizes < K))
    b2 = batch.reshape(N // _RKNN, _RKNN)
    first_b = b2[:, 0]
    last_b = b2[:, -1]
    c0 = jnp.minimum((seg_start[first_b] // 8) * 8, N - _CW)
    nch = (seg_end[last_b] - c0 + _CW - 1) // _CW
    c0 = jnp.where(degenerate, 0, c0).astype(jnp.int32)
    nch = jnp.where(degenerate, N // _CW, nch).astype(jnp.int32)
    return c0, nch


_NC = 2
_NS = 16
_NW = _NC * _NS
_CH = 256


def _sc_gather(table, idx, d, dw=None):
    dw = d if dw is None else dw
    e = idx.shape[0]
    per_w = e // _NW
    n_ch = per_w // _CH
    mesh = plsc.VectorSubcoreMesh(core_axis_name="c", subcore_axis_name="s")

    @functools.partial(
        pl.kernel,
        out_type=jax.ShapeDtypeStruct((e, dw), jnp.float32),
        mesh=mesh,
        scratch_types=[
            pltpu.VMEM((_CH,), jnp.int32),
            pltpu.VMEM((_CH,), jnp.int32),
            pltpu.VMEM((_CH, d), jnp.float32),
            pltpu.VMEM((_CH, d), jnp.float32),
            pltpu.SemaphoreType.DMA,
            pltpu.SemaphoreType.DMA,
        ],
    )
    def gk(table_hbm, idx_hbm, out_hbm, idx0, idx1, rows0, rows1, gsem, osem):
        wid = lax.axis_index("s") * _NC + lax.axis_index("c")
        base = wid * per_w
        wb = (lambda rv: rv) if dw == d else (
            lambda rv: rv.at[:, pl.ds(0, dw)])

        def body(t2, carry):
            for b, (iv, rv) in enumerate(((idx0, rows0), (idx1, rows1))):
                off = base + (t2 * 2 + b) * _CH

                @pl.when(t2 >= 1)
                def _(rv=rv, off=off):
                    pltpu.make_async_copy(
                        wb(rv), out_hbm.at[pl.ds(off, _CH)], osem).wait()

                pltpu.sync_copy(idx_hbm.at[pl.ds(off, _CH)], iv)
                pltpu.async_copy(table_hbm.at[iv], rv, gsem).wait()
                pltpu.async_copy(wb(rv), out_hbm.at[pl.ds(off, _CH)], osem)
            return carry

        lax.fori_loop(0, n_ch // 2, body, 0)
        pltpu.make_async_copy(wb(rows0), out_hbm.at[pl.ds(base, _CH)],
                              osem).wait()
        pltpu.make_async_copy(wb(rows1), out_hbm.at[pl.ds(base, _CH)],
                              osem).wait()

    return gk(table, idx)


_RMLP = 512
_E1 = K * N


def _ag_body(p_ref, wd_ref, wb_ref, b1_ref, a_ref, g_ref, d2_ref):
    p = p_ref[:]
    a_ref[:] = jnp.dot(p, wd_ref[:], preferred_element_type=jnp.float32) + b1_ref[:]
    g_ref[:] = jnp.dot(p, wb_ref[:], preferred_element_type=jnp.float32)
    d2_ref[:] = jnp.sum(p * p, axis=1, keepdims=True)


def _ag(pos, wd, wb, b1):
    return pl.pallas_call(
        _ag_body,
        out_shape=(
            jax.ShapeDtypeStruct((N, 64), jnp.float32),
            jax.ShapeDtypeStruct((N, 128), jnp.float32),
            jax.ShapeDtypeStruct((N, 1), jnp.float32),
        ),
    )(pos, wd, jnp.concatenate([wb, jnp.zeros((3, 64), jnp.float32)], axis=1),
      b1.reshape(1, 64))


def _bn_coefs(st_ref, gamma_ref, beta_ref, count):
    mu = st_ref[0:1, :] * (1.0 / count)
    var = st_ref[1:2, :] * (1.0 / count) - mu * mu
    scale = gamma_ref[:] / jnp.sqrt(var + 1e-5)
    shift = beta_ref[:] - mu * scale
    return scale, shift


def _p2_body(g_ref, a_ref, ga_ref, be_ref, w_ref, b_ref, h1_ref, s_ref,
             s2_ref):
    p = pl.program_id(0)
    i = pl.program_id(1)
    h0 = g_ref[:, :, 0:64] + a_ref[:][None]

    @pl.when(p == 0)
    def _():
        @pl.when(i == 0)
        def _():
            s_ref[:] = jnp.zeros_like(s_ref)

        t = jnp.sum(h0, axis=0)
        u = jnp.sum(h0 * h0, axis=0)
        s_ref[0:1, :] += jnp.sum(t, axis=0, keepdims=True)
        s_ref[1:2, :] += jnp.sum(u, axis=0, keepdims=True)

    @pl.when(p == 1)
    def _():
        @pl.when(i == 0)
        def _():
            s2_ref[:] = jnp.zeros_like(s2_ref)

        scale, shift = _bn_coefs(s_ref, ga_ref, be_ref, _E1)
        y = jnp.maximum(h0 * scale[None] + shift[None], 0.0)
        y2 = y.reshape(K * _RMLP, 64)
        h1 = jnp.dot(y2, w_ref[:], preferred_element_type=jnp.float32) + b_ref[:]
        h1_ref[:] = h1.reshape(K, _RMLP, 64)
        s2_ref[0:1, :] += jnp.sum(h1, axis=0, keepdims=True)
        s2_ref[1:2, :] += jnp.sum(h1 * h1, axis=0, keepdims=True)


def _p2(g3, a, gamma, beta, w2, b2):
    return pl.pallas_call(
        _p2_body,
        grid=(2, N // _RMLP),
        in_specs=[
            pl.BlockSpec((K, _RMLP, 128), lambda p, i: (0, i, 0)),
            pl.BlockSpec((_RMLP, 64), lambda p, i: (i, 0)),
            pl.BlockSpec((1, 64), lambda p, i: (0, 0)),
            pl.BlockSpec((1, 64), lambda p, i: (0, 0)),
            pl.BlockSpec((64, 64), lambda p, i: (0, 0)),
            pl.BlockSpec((1, 64), lambda p, i: (0, 0)),
        ],
        out_specs=(
            pl.BlockSpec((K, _RMLP, 64), lambda p, i: (0, i * p, 0)),
            pl.BlockSpec((8, 64), lambda p, i: (0, 0)),
            pl.BlockSpec((8, 64), lambda p, i: (0, 0)),
        ),
        out_shape=(
            jax.ShapeDtypeStruct((K, N, 64), jnp.float32),
            jax.ShapeDtypeStruct((8, 64), jnp.float32),
            jax.ShapeDtypeStruct((8, 64), jnp.float32),
        ),
    )(g3, a, gamma.reshape(1, 64), beta.reshape(1, 64), w2, b2.reshape(1, 64))


def _p3_body(h1_ref, st_ref, ga_ref, be_ref, w_ref, b_ref, wz_ref, bz_ref,
             wy_ref, x1_ref, z_ref, y_ref, d2_ref):
    scale, shift = _bn_coefs(st_ref, ga_ref, be_ref, _E1)
    yact = jnp.maximum(h1_ref[:] * scale[None] + shift[None], 0.0)
    y2 = yact.reshape(K * _RMLP, 64)
    h2 = jnp.dot(y2, w_ref[:], preferred_element_type=jnp.float32) + b_ref[:]
    x1 = jnp.max(h2.reshape(K, _RMLP, 64), axis=0)
    x1_ref[:] = x1
    z_ref[:] = jnp.dot(x1, wz_ref[:], preferred_element_type=jnp.float32) + bz_ref[:]
    y_ref[:] = jnp.dot(x1, wy_ref[:], preferred_element_type=jnp.float32)
    d2_ref[:] = jnp.sum(x1 * x1, axis=1, keepdims=True)


def _p3(h1, st, gamma, beta, w3, b3, wz, bz, wy):
    return pl.pallas_call(
        _p3_body,
        grid=(N // _RMLP,),
        in_specs=[
            pl.BlockSpec((K, _RMLP, 64), lambda i: (0, i, 0)),
            pl.BlockSpec((8, 64), lambda i: (0, 0)),
            pl.BlockSpec((1, 64), lambda i: (0, 0)),
            pl.BlockSpec((1, 64), lambda i: (0, 0)),
            pl.BlockSpec((64, 64), lambda i: (0, 0)),
            pl.BlockSpec((1, 64), lambda i: (0, 0)),
            pl.BlockSpec((64, 128), lambda i: (0, 0)),
            pl.BlockSpec((1, 128), lambda i: (0, 0)),
            pl.BlockSpec((64, 128), lambda i: (0, 0)),
        ],
        out_specs=(
            pl.BlockSpec((_RMLP, 64), lambda i: (i, 0)),
            pl.BlockSpec((_RMLP, 128), lambda i: (i, 0)),
            pl.BlockSpec((_RMLP, 128), lambda i: (i, 0)),
            pl.BlockSpec((_RMLP, 1), lambda i: (i, 0)),
        ),
        out_shape=(
            jax.ShapeDtypeStruct((N, 64), jnp.float32),
            jax.ShapeDtypeStruct((N, 128), jnp.float32),
            jax.ShapeDtypeStruct((N, 128), jnp.float32),
            jax.ShapeDtypeStruct((N, 1), jnp.float32),
        ),
    )(h1, st, gamma.reshape(1, 64), beta.reshape(1, 64), w3, b3.reshape(1, 64),
      wz, bz.reshape(1, 128), wy)


_NPB = N // _RMLP


def _pool_body(x1_ref, yg_ref, z_ref, bt_ref, wl_ref, bl_ref,
               wm1_ref, bm1_ref, wm2_ref, bm2_ref, wm3_ref, bm3_ref,
               o_ref, p_ref):
    i = pl.program_id(0)

    @pl.when(i == 0)
    def _():
        p_ref[:] = jnp.full_like(p_ref, -jnp.inf)

    @pl.when(i < _NPB)
    def _():
        x2 = z_ref[:] + jnp.max(yg_ref[:], axis=0)
        xc = jnp.concatenate([x1_ref[:], x2], axis=1)
        out = jnp.dot(xc, wl_ref[:], preferred_element_type=jnp.float32) + bl_ref[:]
        bt = bt_ref[:]
        for b in range(B):
            mb = jnp.max(jnp.where(bt == b, out, -jnp.inf), axis=0,
                         keepdims=True)
            p_ref[pl.ds(b, 1), :] = jnp.maximum(p_ref[pl.ds(b, 1), :], mb)

    @pl.when(i == _NPB)
    def _():
        h = jnp.maximum(
            jnp.dot(p_ref[:], wm1_ref[:],
                    preferred_element_type=jnp.float32) + bm1_ref[:], 0.0)
        h = jnp.maximum(
            jnp.dot(h, wm2_ref[:],
                    preferred_element_type=jnp.float32) + bm2_ref[:], 0.0)
        h = jnp.dot(h, wm3_ref[:], preferred_element_type=jnp.float32) + bm3_ref[:]
        m = jnp.max(h, axis=1, keepdims=True)
        s = h - m
        o_ref[:] = s - jnp.log(jnp.sum(jnp.exp(s), axis=1, keepdims=True))


def _pool_head(x1, yg3, z, bc, wl, bl, wm1, bm1, wm2, bm2, wm3, bm3):
    blk = lambda i: jnp.minimum(i, _NPB - 1)
    return pl.pallas_call(
        _pool_body,
        grid=(_NPB + 1,),
        in_specs=[
            pl.BlockSpec((_RMLP, 64), lambda i: (blk(i), 0)),
            pl.BlockSpec((K, _RMLP, 128), lambda i: (0, blk(i), 0)),
            pl.BlockSpec((_RMLP, 128), lambda i: (blk(i), 0)),
            pl.BlockSpec((_RMLP, 1), lambda i: (blk(i), 0)),
            pl.BlockSpec((192, 1024), lambda i: (0, 0)),
            pl.BlockSpec((1, 1024), lambda i: (0, 0)),
            pl.BlockSpec((1024, 512), lambda i: (0, 0)),
            pl.BlockSpec((1, 512), lambda i: (0, 0)),
            pl.BlockSpec((512, 256), lambda i: (0, 0)),
            pl.BlockSpec((1, 256), lambda i: (0, 0)),
            pl.BlockSpec((256, 40), lambda i: (0, 0)),
            pl.BlockSpec((1, 40), lambda i: (0, 0)),
        ],
        out_specs=pl.BlockSpec((B, 40), lambda i: (0, 0)),
        out_shape=jax.ShapeDtypeStruct((B, 40), jnp.float32),
        scratch_shapes=[pltpu.VMEM((B, 1024), jnp.float32)],
    )(x1, yg3, z, bc, wl, bl.reshape(1, 1024), wm1, bm1.reshape(1, 512),
      wm2, bm2.reshape(1, 256), wm3, bm3.reshape(1, 40))


def kernel(pos, batch, W1, b1, g1, be1, W2, b2, g2, be2, W3, b3, Wc2, bc2,
           Wl, bl, Wm1, bm1, Wm2, bm2, Wm3, bm3):
    batch = batch.astype(jnp.int32)
    bc = batch.reshape(N, 1)
    br = batch.reshape(1, N)
    c0s, nchs = _seg_ranges(batch)

    a1, g1t, d2p = _ag(pos, W1[:3] - W1[3:], W1[3:], b1)
    idx1 = _knn(pos, d2p, bc, br, c0s, nchs)
    gat = _sc_gather(g1t, idx1.reshape(K * N), 128)
    g3 = gat.reshape(K, N, 128)
    h1, st1, st2 = _p2(g3, a1, g1, be1, W2, b2)
    x1, z, y, d2x = _p3(h1, st2, g2, be2, W3, b3,
                        Wc2[:64] - Wc2[64:], bc2, Wc2[64:])

    idx2 = _knn(x1, d2x, bc, br, c0s, nchs)
    yg = _sc_gather(y, idx2.reshape(K * N), 128)

    return _pool_head(x1, yg.reshape(K, N, 128), z, bc, Wl, bl,
                      Wm1, bm1, Wm2, bm2, Wm3, bm3)


# speedup vs baseline: 20.5570x; 1.0053x over previous
import functools

import jax
import jax.numpy as jnp
from jax import lax
from jax.experimental import pallas as pl
from jax.experimental.pallas import tpu as pltpu
from jax.experimental.pallas import tpu_sc as plsc

N = 8192
B = 8
K = 20
BIG = 1e30
BIG2 = 2e30


_RKNN = 256
_CW = 512
_IBIG = 2**30


def _knn_body(c0_ref, nch_ref, x_ref, xr_ref, d2c_ref, bc_ref, br_ref,
              idx_ref, dist_ref):
    i = pl.program_id(0)
    c0 = c0_ref[i]
    nch = nch_ref[i]
    br = br_ref[:]
    xr = xr_ref[:]
    dx = xr.shape[1]
    d2r = lax.dot_general(
        jnp.ones((1, dx), jnp.float32), xr * xr,
        dimension_numbers=(((1,), (1,)), ((), ())),
        preferred_element_type=jnp.float32,
    )
    rowi0 = lax.broadcasted_iota(jnp.int32, (_CW, _RKNN), 0)
    minf = jnp.full((1, _RKNN), jnp.inf, jnp.float32)
    ibig = jnp.full((1, _RKNN), _IBIG, jnp.int32)

    def _scan_chunk(d, off, m, ji):
        lm = jnp.min(d, axis=0, keepdims=True)
        la = jnp.min(jnp.where(d == lm, rowi0, jnp.int32(_IBIG)),
                     axis=0, keepdims=True) + off
        better = lm < m
        return jnp.minimum(m, lm), jnp.where(better, la, ji)

    def fill(t, mj):
        m, ji = mj
        start = jnp.minimum(c0 + t * _CW, N - _CW)
        mm = lax.dot_general(
            x_ref[pl.ds(start, _CW), :], xr,
            dimension_numbers=(((1,), (1,)), ((), ())),
            preferred_element_type=jnp.float32,
            precision=lax.Precision.DEFAULT,
        )
        d = d2c_ref[pl.ds(start, _CW), :] + d2r - 2.0 * mm
        d = jnp.where(bc_ref[pl.ds(start, _CW), :] != br, BIG, d)
        dist_ref[pl.ds(t * _CW, _CW), :] = d
        return _scan_chunk(d, start, m, ji)

    m, ji = lax.fori_loop(0, nch, fill, (minf, ibig))

    picks = [ji]
    for step in range(1, K):
        prev = ji
        last = step == K - 1

        def ext(t, mj, prev=prev, last=last):
            m2, ji2 = mj
            off = jnp.minimum(c0 + t * _CW, N - _CW)
            d = dist_ref[pl.ds(t * _CW, _CW), :]
            d = jnp.where(rowi0 == (prev - off), BIG2, d)
            if not last:
                dist_ref[pl.ds(t * _CW, _CW), :] = d
            return _scan_chunk(d, off, m2, ji2)

        m, ji = lax.fori_loop(0, nch, ext, (minf, ibig))
        picks.append(ji)
    idx_ref[:] = jnp.concatenate(picks, axis=0)


def _knn(x, d2c, bc, br, c0s, nchs):
    dx = x.shape[1]
    grid_spec = pltpu.PrefetchScalarGridSpec(
        num_scalar_prefetch=2,
        grid=(N // _RKNN,),
        in_specs=[
            pl.BlockSpec((N, dx), lambda i, *_: (0, 0)),
            pl.BlockSpec((_RKNN, dx), lambda i, *_: (i, 0)),
            pl.BlockSpec((N, 1), lambda i, *_: (0, 0)),
            pl.BlockSpec((N, 1), lambda i, *_: (0, 0)),
            pl.BlockSpec((1, _RKNN), lambda i, *_: (0, i)),
        ],
        out_specs=pl.BlockSpec((K, _RKNN), lambda i, *_: (0, i)),
        scratch_shapes=[pltpu.VMEM((N, _RKNN), jnp.float32)],
    )
    return pl.pallas_call(
        _knn_body,
        grid_spec=grid_spec,
        out_shape=jax.ShapeDtypeStruct((K, N), jnp.int32),
    )(c0s, nchs, x, x, d2c, bc, br)


def _seg_ranges(batch):
    ar = jnp.arange(B, dtype=batch.dtype)
    seg_start = jnp.searchsorted(batch, ar).astype(jnp.int32)
    seg_end = jnp.searchsorted(batch, ar, side="right").astype(jnp.int32)
    sizes = seg_end - seg_start
    degenerate = jnp.any((sizes > 0) & (sizes < K))
    b2 = batch.reshape(N // _RKNN, _RKNN)
    first_b = b2[:, 0]
    last_b = b2[:, -1]
    c0 = jnp.minimum((seg_start[first_b] // 8) * 8, N - _CW)
    nch = (seg_end[last_b] - c0 + _CW - 1) // _CW
    c0 = jnp.where(degenerate, 0, c0).astype(jnp.int32)
    nch = jnp.where(degenerate, N // _CW, nch).astype(jnp.int32)
    return c0, nch


_NC = 2
_NS = 16
_NW = _NC * _NS
_CH = 256


def _sc_gather(table, idx, d, dw=None):
    dw = d if dw is None else dw
    e = idx.shape[0]
    per_w = e // _NW
    n_ch = per_w // _CH
    mesh = plsc.VectorSubcoreMesh(core_axis_name="c", subcore_axis_name="s")

    @functools.partial(
        pl.kernel,
        out_type=jax.ShapeDtypeStruct((e, dw), jnp.float32),
        mesh=mesh,
        scratch_types=[
            pltpu.VMEM((_CH,), jnp.int32),
            pltpu.VMEM((_CH,), jnp.int32),
            pltpu.VMEM((_CH, d), jnp.float32),
            pltpu.VMEM((_CH, d), jnp.float32),
            pltpu.SemaphoreType.DMA,
            pltpu.SemaphoreType.DMA,
        ],
    )
    def gk(table_hbm, idx_hbm, out_hbm, idx0, idx1, rows0, rows1, gsem, osem):
        wid = lax.axis_index("s") * _NC + lax.axis_index("c")
        base = wid * per_w
        wb = (lambda rv: rv) if dw == d else (
            lambda rv: rv.at[:, pl.ds(0, dw)])

        def body(t2, carry):
            for b, (iv, rv) in enumerate(((idx0, rows0), (idx1, rows1))):
                off = base + (t2 * 2 + b) * _CH

                @pl.when(t2 >= 1)
                def _(rv=rv, off=off):
                    pltpu.make_async_copy(
                        wb(rv), out_hbm.at[pl.ds(off, _CH)], osem).wait()

                pltpu.sync_copy(idx_hbm.at[pl.ds(off, _CH)], iv)
                pltpu.async_copy(table_hbm.at[iv], rv, gsem).wait()
                pltpu.async_copy(wb(rv), out_hbm.at[pl.ds(off, _CH)], osem)
            return carry

        lax.fori_loop(0, n_ch // 2, body, 0)
        pltpu.make_async_copy(wb(rows0), out_hbm.at[pl.ds(base, _CH)],
                              osem).wait()
        pltpu.make_async_copy(wb(rows1), out_hbm.at[pl.ds(base, _CH)],
                              osem).wait()

    return gk(table, idx)


_RMLP = 512
_E1 = K * N


def _ag_body(p_ref, wd_ref, wb_ref, b1_ref, a_ref, g_ref, d2_ref):
    p = p_ref[:]
    a_ref[:] = jnp.dot(p, wd_ref[:], preferred_element_type=jnp.float32) + b1_ref[:]
    g_ref[:] = jnp.dot(p, wb_ref[:], preferred_element_type=jnp.float32)
    d2_ref[:] = jnp.sum(p * p, axis=1, keepdims=True)


def _ag(pos, wd, wb, b1):
    return pl.pallas_call(
        _ag_body,
        out_shape=(
            jax.ShapeDtypeStruct((N, 64), jnp.float32),
            jax.ShapeDtypeStruct((N, 128), jnp.float32),
            jax.ShapeDtypeStruct((N, 1), jnp.float32),
        ),
    )(pos, wd, jnp.concatenate([wb, jnp.zeros((3, 64), jnp.float32)], axis=1),
      b1.reshape(1, 64))


def _bn_coefs(st_ref, gamma_ref, beta_ref, count):
    mu = st_ref[0:1, :] * (1.0 / count)
    var = st_ref[1:2, :] * (1.0 / count) - mu * mu
    scale = gamma_ref[:] / jnp.sqrt(var + 1e-5)
    shift = beta_ref[:] - mu * scale
    return scale, shift


def _p2_body(g_ref, a_ref, ga_ref, be_ref, w_ref, b_ref, h1_ref, s_ref,
             s2_ref):
    p = pl.program_id(0)
    i = pl.program_id(1)
    h0 = g_ref[:, :, 0:64] + a_ref[:][None]

    @pl.when(p == 0)
    def _():
        @pl.when(i == 0)
        def _():
            s_ref[:] = jnp.zeros_like(s_ref)

        t = jnp.sum(h0, axis=0)
        u = jnp.sum(h0 * h0, axis=0)
        s_ref[0:1, :] += jnp.sum(t, axis=0, keepdims=True)
        s_ref[1:2, :] += jnp.sum(u, axis=0, keepdims=True)

    @pl.when(p == 1)
    def _():
        @pl.when(i == 0)
        def _():
            s2_ref[:] = jnp.zeros_like(s2_ref)

        scale, shift = _bn_coefs(s_ref, ga_ref, be_ref, _E1)
        y = jnp.maximum(h0 * scale[None] + shift[None], 0.0)
        y2 = y.reshape(K * _RMLP, 64)
        h1 = jnp.dot(y2, w_ref[:], preferred_element_type=jnp.float32) + b_ref[:]
        h1_ref[:] = h1.reshape(K, _RMLP, 64)
        s2_ref[0:1, :] += jnp.sum(h1, axis=0, keepdims=True)
        s2_ref[1:2, :] += jnp.sum(h1 * h1, axis=0, keepdims=True)


def _p2(g3, a, gamma, beta, w2, b2):
    return pl.pallas_call(
        _p2_body,
        grid=(2, N // _RMLP),
        in_specs=[
            pl.BlockSpec((K, _RMLP, 128), lambda p, i: (0, i, 0)),
            pl.BlockSpec((_RMLP, 64), lambda p, i: (i, 0)),
            pl.BlockSpec((1, 64), lambda p, i: (0, 0)),
            pl.BlockSpec((1, 64), lambda p, i: (0, 0)),
            pl.BlockSpec((64, 64), lambda p, i: (0, 0)),
            pl.BlockSpec((1, 64), lambda p, i: (0, 0)),
        ],
        out_specs=(
            pl.BlockSpec((K, _RMLP, 64), lambda p, i: (0, i * p, 0)),
            pl.BlockSpec((8, 64), lambda p, i: (0, 0)),
            pl.BlockSpec((8, 64), lambda p, i: (0, 0)),
        ),
        out_shape=(
            jax.ShapeDtypeStruct((K, N, 64), jnp.float32),
            jax.ShapeDtypeStruct((8, 64), jnp.float32),
            jax.ShapeDtypeStruct((8, 64), jnp.float32),
        ),
    )(g3, a, gamma.reshape(1, 64), beta.reshape(1, 64), w2, b2.reshape(1, 64))


def _p3_body(h1_ref, st_ref, ga_ref, be_ref, w_ref, b_ref, wz_ref, bz_ref,
             wy_ref, x1_ref, z_ref, y_ref, d2_ref):
    scale, shift = _bn_coefs(st_ref, ga_ref, be_ref, _E1)
    yact = jnp.maximum(h1_ref[:] * scale[None] + shift[None], 0.0)
    y2 = yact.reshape(K * _RMLP, 64)
    h2 = jnp.dot(y2, w_ref[:], preferred_element_type=jnp.float32) + b_ref[:]
    x1 = jnp.max(h2.reshape(K, _RMLP, 64), axis=0)
    x1_ref[:] = x1
    z_ref[:] = jnp.dot(x1, wz_ref[:], preferred_element_type=jnp.float32) + bz_ref[:]
    y_ref[:] = jnp.dot(x1, wy_ref[:], preferred_element_type=jnp.float32)
    d2_ref[:] = jnp.sum(x1 * x1, axis=1, keepdims=True)


def _p3(h1, st, gamma, beta, w3, b3, wz, bz, wy):
    return pl.pallas_call(
        _p3_body,
        grid=(N // _RMLP,),
        in_specs=[
            pl.BlockSpec((K, _RMLP, 64), lambda i: (0, i, 0)),
            pl.BlockSpec((8, 64), lambda i: (0, 0)),
            pl.BlockSpec((1, 64), lambda i: (0, 0)),
            pl.BlockSpec((1, 64), lambda i: (0, 0)),
            pl.BlockSpec((64, 64), lambda i: (0, 0)),
            pl.BlockSpec((1, 64), lambda i: (0, 0)),
            pl.BlockSpec((64, 128), lambda i: (0, 0)),
            pl.BlockSpec((1, 128), lambda i: (0, 0)),
            pl.BlockSpec((64, 128), lambda i: (0, 0)),
        ],
        out_specs=(
            pl.BlockSpec((_RMLP, 64), lambda i: (i, 0)),
            pl.BlockSpec((_RMLP, 128), lambda i: (i, 0)),
            pl.BlockSpec((_RMLP, 128), lambda i: (i, 0)),
            pl.BlockSpec((_RMLP, 1), lambda i: (i, 0)),
        ),
        out_shape=(
            jax.ShapeDtypeStruct((N, 64), jnp.float32),
            jax.ShapeDtypeStruct((N, 128), jnp.float32),
            jax.ShapeDtypeStruct((N, 128), jnp.float32),
            jax.ShapeDtypeStruct((N, 1), jnp.float32),
        ),
    )(h1, st, gamma.reshape(1, 64), beta.reshape(1, 64), w3, b3.reshape(1, 64),
      wz, bz.reshape(1, 128), wy)


_NPB = N // _RMLP


def _pool_body(fb_ref, lb_ref, x1_ref, yg_ref, z_ref, bt_ref, wl_ref, bl_ref,
               wm1_ref, bm1_ref, wm2_ref, bm2_ref, wm3_ref, bm3_ref,
               o_ref, p_ref):
    i = pl.program_id(0)

    @pl.when(i == 0)
    def _():
        p_ref[:] = jnp.full_like(p_ref, -jnp.inf)

    @pl.when(i < _NPB)
    def _():
        x2 = z_ref[:] + jnp.max(yg_ref[:], axis=0)
        xc = jnp.concatenate([x1_ref[:], x2], axis=1)
        out = jnp.dot(xc, wl_ref[:], preferred_element_type=jnp.float32) + bl_ref[:]
        bt = bt_ref[:]
        ic = jnp.minimum(i, _NPB - 1)

        def seg(b, carry):
            mb = jnp.max(jnp.where(bt == b, out, -jnp.inf), axis=0,
                         keepdims=True)
            p_ref[pl.ds(b, 1), :] = jnp.maximum(p_ref[pl.ds(b, 1), :], mb)
            return carry

        lax.fori_loop(fb_ref[ic], lb_ref[ic] + 1, seg, 0)

    @pl.when(i == _NPB)
    def _():
        h = jnp.maximum(
            jnp.dot(p_ref[:], wm1_ref[:],
                    preferred_element_type=jnp.float32) + bm1_ref[:], 0.0)
        h = jnp.maximum(
            jnp.dot(h, wm2_ref[:],
                    preferred_element_type=jnp.float32) + bm2_ref[:], 0.0)
        h = jnp.dot(h, wm3_ref[:], preferred_element_type=jnp.float32) + bm3_ref[:]
        m = jnp.max(h, axis=1, keepdims=True)
        s = h - m
        o_ref[:] = s - jnp.log(jnp.sum(jnp.exp(s), axis=1, keepdims=True))


def _pool_head(x1, yg3, z, bc, fb, lb, wl, bl, wm1, bm1, wm2, bm2, wm3, bm3):
    blk = lambda i: jnp.minimum(i, _NPB - 1)
    grid_spec = pltpu.PrefetchScalarGridSpec(
        num_scalar_prefetch=2,
        grid=(_NPB + 1,),
        in_specs=[
            pl.BlockSpec((_RMLP, 64), lambda i, *_: (blk(i), 0)),
            pl.BlockSpec((K, _RMLP, 128), lambda i, *_: (0, blk(i), 0)),
            pl.BlockSpec((_RMLP, 128), lambda i, *_: (blk(i), 0)),
            pl.BlockSpec((_RMLP, 1), lambda i, *_: (blk(i), 0)),
            pl.BlockSpec((192, 1024), lambda i, *_: (0, 0)),
            pl.BlockSpec((1, 1024), lambda i, *_: (0, 0)),
            pl.BlockSpec((1024, 512), lambda i, *_: (0, 0)),
            pl.BlockSpec((1, 512), lambda i, *_: (0, 0)),
            pl.BlockSpec((512, 256), lambda i, *_: (0, 0)),
            pl.BlockSpec((1, 256), lambda i, *_: (0, 0)),
            pl.BlockSpec((256, 40), lambda i, *_: (0, 0)),
            pl.BlockSpec((1, 40), lambda i, *_: (0, 0)),
        ],
        out_specs=pl.BlockSpec((B, 40), lambda i, *_: (0, 0)),
        scratch_shapes=[pltpu.VMEM((B, 1024), jnp.float32)],
    )
    return pl.pallas_call(
        _pool_body,
        grid_spec=grid_spec,
        out_shape=jax.ShapeDtypeStruct((B, 40), jnp.float32),
    )(fb, lb, x1, yg3, z, bc, wl, bl.reshape(1, 1024), wm1,
      bm1.reshape(1, 512), wm2, bm2.reshape(1, 256), wm3, bm3.reshape(1, 40))


def kernel(pos, batch, W1, b1, g1, be1, W2, b2, g2, be2, W3, b3, Wc2, bc2,
           Wl, bl, Wm1, bm1, Wm2, bm2, Wm3, bm3):
    batch = batch.astype(jnp.int32)
    bc = batch.reshape(N, 1)
    br = batch.reshape(1, N)
    c0s, nchs = _seg_ranges(batch)

    a1, g1t, d2p = _ag(pos, W1[:3] - W1[3:], W1[3:], b1)
    idx1 = _knn(pos, d2p, bc, br, c0s, nchs)
    gat = _sc_gather(g1t, idx1.reshape(K * N), 128)
    g3 = gat.reshape(K, N, 128)
    h1, st1, st2 = _p2(g3, a1, g1, be1, W2, b2)
    x1, z, y, d2x = _p3(h1, st2, g2, be2, W3, b3,
                        Wc2[:64] - Wc2[64:], bc2, Wc2[64:])

    idx2 = _knn(x1, d2x, bc, br, c0s, nchs)
    yg = _sc_gather(y, idx2.reshape(K * N), 128)

    bp = batch.reshape(_NPB, _RMLP)
    return _pool_head(x1, yg.reshape(K, N, 128), z, bc,
                      bp[:, 0], bp[:, -1], Wl, bl,
                      Wm1, bm1, Wm2, bm2, Wm3, bm3)
